# baseline (speedup 1.0000x reference)
import functools

import jax
import jax.numpy as jnp
from jax import lax
from jax.experimental import pallas as pl
from jax.experimental.pallas import tpu as pltpu
from jax.experimental.pallas import tpu_sc as plsc

B = 4096
D = 64
_NC = 2
_NS = 16
_NW = _NC * _NS
_BPW = B // _NW

_F32 = jnp.float32


def _dot_t(a, b):
    return lax.dot_general(a, b, (((1,), (1,)), ((), ())),
                           preferred_element_type=_F32)


def _dot(a, b):
    return lax.dot_general(a, b, (((1,), (0,)), ((), ())),
                           preferred_element_type=_F32)


def _silu(x):
    return x / (1.0 + jnp.exp(-x))


def _l2norm(x):
    n = jnp.sqrt(jnp.sum(x * x, axis=-1, keepdims=True))
    return x / jnp.maximum(n, 1e-16)


_HALF = B // 2
_CH = 512


def _sc_gather3(idx_a, idx_c, idx_b, T_a, T_c, T_b):
    mesh = plsc.ScalarSubcoreMesh(axis_name="c", num_cores=_NC)
    out_t = jax.ShapeDtypeStruct((B, D), _F32)

    @functools.partial(
        pl.kernel,
        out_type=[out_t, out_t, out_t],
        mesh=mesh,
        scratch_types=[
            pltpu.SMEM((_CH,), jnp.int32),
            pltpu.SemaphoreType.DMA,
        ],
    )
    def gather_k(ia_h, ic_h, ib_h, Ta_h, Tc_h, Tb_h, oa_h, oc_h, ob_h,
                 ism, sem):
        cid = lax.axis_index("c")
        base = cid * _HALF
        for idx_h, T_h, o_h in ((ia_h, Ta_h, oa_h), (ic_h, Tc_h, oc_h),
                                (ib_h, Tb_h, ob_h)):
            for ch in range(_HALF // _CH):
                off = base + ch * _CH
                pltpu.sync_copy(idx_h.at[pl.ds(off, _CH)], ism)

                def issue(j, carry, idx_s=ism, T=T_h, o=o_h, off=off):
                    pltpu.async_copy(T.at[pl.ds(idx_s[j], 1)],
                                     o.at[pl.ds(off + j, 1)], sem)
                    return carry

                lax.fori_loop(0, _CH, issue, 0)
        for T_h, o_h in ((Ta_h, oa_h), (Tc_h, oc_h), (Tb_h, ob_h)):
            pltpu.make_async_copy(T_h.at[pl.ds(0, _HALF)],
                                  o_h.at[pl.ds(base, _HALF)], sem).wait()

    return gather_k(idx_a, idx_c, idx_b, T_a, T_c, T_b)


_R_MLP = 512


def _ad_mlp_body(ea, ec, eb, w1a, w1c, w1b, b1, w2, b2, w3, b3, out):
    h = _dot_t(ea[...], w1a[...]) + _dot_t(ec[...], w1c[...]) \
        + _dot_t(eb[...], w1b[...]) + b1[...]
    h = _silu(h)
    h = _silu(_dot_t(h, w2[...]) + b2[...])
    h = _dot_t(h, w3[...]) + b3[...]
    out[...] = _l2norm(_l2norm(h))


def _ad_mlp(ea, ec, eb, Wa1, ba1, Wa2, ba2, Wa3, ba3):
    full = lambda shape: pl.BlockSpec(shape, lambda i: (0, 0))
    row = pl.BlockSpec((_R_MLP, D), lambda i: (i, 0))
    return pl.pallas_call(
        _ad_mlp_body,
        grid=(B // _R_MLP,),
        in_specs=[
            row, row, row,
            full((256, D)), full((256, D)), full((256, D)), full((1, 256)),
            full((128, 256)), full((1, 128)),
            full((64, 128)), full((1, 64)),
        ],
        out_specs=row,
        out_shape=jax.ShapeDtypeStruct((B, D), _F32),
    )(ea, ec, eb, Wa1[:, 0:D], Wa1[:, D:2 * D], Wa1[:, 2 * D:3 * D],
      ba1.reshape(1, -1), Wa2, ba2.reshape(1, -1), Wa3, ba3.reshape(1, -1))


_R_LOSS = 256
_NB_LOSS = B // _R_LOSS


def _loss_body(ad_emb, u_row, ts_row, clk_row, ids_row, qp_row,
               u_col, ts_col, clk_col, ids_col,
               w1, b1, w2, b2, w3, b3, out, acc):
    i = pl.program_id(0)

    @pl.when(i == 0)
    def _init():
        acc[0] = 0.0
        acc[1] = 0.0

    ad = ad_emb[...]
    clicked = clk_row[...] == 1
    same_user = u_col[...] == u_row[...]
    causal = ts_col[...] > ts_row[...]
    maskf = (clicked & same_user & causal).astype(_F32)
    msum = jnp.sum(maskf, axis=1, keepdims=True)
    hist = _dot(maskf, ad) / (msum + 1e-16)
    hist_n2 = jnp.sum(hist * hist, axis=1, keepdims=True)
    g = _silu(_dot_t(hist, w1[...]) + b1[...])
    g = _silu(_dot_t(g, w2[...]) + b2[...])
    g = _dot_t(g, w3[...]) + b3[...]
    x = _l2norm(_l2norm(g))
    user_emb = jnp.where(hist_n2 == 0.0, 0.0, x)
    u_n2 = jnp.sum(user_emb * user_emb, axis=1, keepdims=True)
    validf = ((clk_col[...] == 1) & (u_n2 != 0.0)).astype(_F32)
    logits = _dot_t(user_emb, ad) - jnp.log(qp_row[...])
    cols = lax.broadcasted_iota(jnp.int32, (_R_LOSS, B), 1)
    rows = lax.broadcasted_iota(jnp.int32, (_R_LOSS, B), 0) + i * _R_LOSS
    eye = cols == rows
    acc_hits = (ids_col[...] == ids_row[...]) & jnp.logical_not(eye)
    logits = jnp.where(acc_hits, -1e9, logits)
    m = jnp.max(logits, axis=1, keepdims=True)
    lse = m + jnp.log(jnp.sum(jnp.exp(logits - m), axis=1, keepdims=True))
    diag = jnp.sum(jnp.where(eye, logits, 0.0), axis=1, keepdims=True)
    pos_logp = diag - lse
    acc[0] += jnp.sum(pos_logp * validf)
    acc[1] += jnp.sum(validf)

    @pl.when(i == _NB_LOSS - 1)
    def _fin():
        out[...] = jnp.broadcast_to(-acc[0] / (acc[1] + 1e-16), (1, 1))


def _loss(ad_emb, user, timestamp, is_click, ad_ids, q_proba,
          Wu1, bu1, Wu2, bu2, Wu3, bu3):
    full = lambda shape: pl.BlockSpec(shape, lambda i: (0, 0))
    col = pl.BlockSpec((_R_LOSS, 1), lambda i: (i, 0))
    u_row = user.reshape(1, B)
    ts_row = timestamp.reshape(1, B)
    clk_row = is_click.reshape(1, B)
    ids_row = ad_ids.reshape(1, B)
    out = pl.pallas_call(
        _loss_body,
        grid=(_NB_LOSS,),
        in_specs=[
            full((B, D)),
            full((1, B)), full((1, B)), full((1, B)), full((1, B)),
            full((1, B)),
            col, col, col, col,
            full((256, D)), full((1, 256)),
            full((128, 256)), full((1, 128)),
            full((64, 128)), full((1, 64)),
        ],
        out_specs=full((1, 1)),
        out_shape=jax.ShapeDtypeStruct((1, 1), _F32),
        scratch_shapes=[pltpu.SMEM((2,), _F32)],
    )(ad_emb, u_row, ts_row, clk_row, ids_row, q_proba.reshape(1, B),
      u_row.reshape(B, 1), ts_row.reshape(B, 1), clk_row.reshape(B, 1),
      ids_row.reshape(B, 1),
      Wu1, bu1.reshape(1, -1), Wu2, bu2.reshape(1, -1),
      Wu3, bu3.reshape(1, -1))
    return out[0, 0]


def kernel(adgroup_id, cate_id, brand, user, timestamp, is_click, q_proba,
           T_adgroup, T_cate, T_brand, Wa1, ba1, Wa2, ba2, Wa3, ba3,
           Wu1, bu1, Wu2, bu2, Wu3, bu3):
    ia = adgroup_id.reshape(B).astype(jnp.int32)
    ic = cate_id.reshape(B).astype(jnp.int32)
    ib = brand.reshape(B).astype(jnp.int32)
    ea, ec, eb = _sc_gather3(ia, ic, ib, T_adgroup, T_cate, T_brand)
    ad_emb = _ad_mlp(ea, ec, eb, Wa1, ba1, Wa2, ba2, Wa3, ba3)
    return _loss(ad_emb, user.reshape(B), timestamp, is_click.astype(jnp.int32),
                 ia, q_proba, Wu1, bu1, Wu2, bu2, Wu3, bu3)


# speedup vs baseline: 1.0017x; 1.0017x over previous
import functools

import jax
import jax.numpy as jnp
from jax import lax
from jax.experimental import pallas as pl
from jax.experimental.pallas import tpu as pltpu
from jax.experimental.pallas import tpu_sc as plsc

B = 4096
D = 64
_NC = 2
_NS = 16
_NW = _NC * _NS
_BPW = B // _NW

_F32 = jnp.float32


def _dot_t(a, b):
    return lax.dot_general(a, b, (((1,), (1,)), ((), ())),
                           preferred_element_type=_F32)


def _dot(a, b):
    return lax.dot_general(a, b, (((1,), (0,)), ((), ())),
                           preferred_element_type=_F32)


def _silu(x):
    return x / (1.0 + jnp.exp(-x))


def _l2norm(x):
    n = jnp.sqrt(jnp.sum(x * x, axis=-1, keepdims=True))
    return x / jnp.maximum(n, 1e-16)


_HALF = B // 2
_CH = 512


def _sc_gather3(idx_a, idx_c, idx_b, T_a, T_c, T_b):
    mesh = plsc.ScalarSubcoreMesh(axis_name="c", num_cores=_NC)
    out_t = jax.ShapeDtypeStruct((B, D), _F32)

    @functools.partial(
        pl.kernel,
        out_type=[out_t, out_t, out_t],
        mesh=mesh,
        scratch_types=[
            pltpu.SMEM((_CH,), jnp.int32),
            pltpu.SemaphoreType.DMA,
        ],
        compiler_params=pltpu.CompilerParams(use_tc_tiling_on_sc=True),
    )
    def gather_k(ia_h, ic_h, ib_h, Ta_h, Tc_h, Tb_h, oa_h, oc_h, ob_h,
                 ism, sem):
        cid = lax.axis_index("c")
        base = cid * _HALF
        for idx_h, T_h, o_h in ((ia_h, Ta_h, oa_h), (ic_h, Tc_h, oc_h),
                                (ib_h, Tb_h, ob_h)):
            for ch in range(_HALF // _CH):
                off = base + ch * _CH
                pltpu.sync_copy(idx_h.at[pl.ds(off, _CH)], ism)

                def issue(j, carry, idx_s=ism, T=T_h, o=o_h, off=off):
                    pltpu.async_copy(T.at[pl.ds(idx_s[j], 1)],
                                     o.at[pl.ds(off + j, 1)], sem)
                    return carry

                lax.fori_loop(0, _CH, issue, 0)
        for T_h, o_h in ((Ta_h, oa_h), (Tc_h, oc_h), (Tb_h, ob_h)):
            pltpu.make_async_copy(T_h.at[pl.ds(0, _HALF)],
                                  o_h.at[pl.ds(base, _HALF)], sem).wait()

    return gather_k(idx_a, idx_c, idx_b, T_a, T_c, T_b)


_R_MLP = 512


def _ad_mlp_body(ea, ec, eb, w1a, w1c, w1b, b1, w2, b2, w3, b3, out):
    h = _dot_t(ea[...], w1a[...]) + _dot_t(ec[...], w1c[...]) \
        + _dot_t(eb[...], w1b[...]) + b1[...]
    h = _silu(h)
    h = _silu(_dot_t(h, w2[...]) + b2[...])
    h = _dot_t(h, w3[...]) + b3[...]
    out[...] = _l2norm(_l2norm(h))


def _ad_mlp(ea, ec, eb, Wa1, ba1, Wa2, ba2, Wa3, ba3):
    full = lambda shape: pl.BlockSpec(shape, lambda i: (0, 0))
    row = pl.BlockSpec((_R_MLP, D), lambda i: (i, 0))
    return pl.pallas_call(
        _ad_mlp_body,
        grid=(B // _R_MLP,),
        in_specs=[
            row, row, row,
            full((256, D)), full((256, D)), full((256, D)), full((1, 256)),
            full((128, 256)), full((1, 128)),
            full((64, 128)), full((1, 64)),
        ],
        out_specs=row,
        out_shape=jax.ShapeDtypeStruct((B, D), _F32),
    )(ea, ec, eb, Wa1[:, 0:D], Wa1[:, D:2 * D], Wa1[:, 2 * D:3 * D],
      ba1.reshape(1, -1), Wa2, ba2.reshape(1, -1), Wa3, ba3.reshape(1, -1))


_R_LOSS = 256
_NB_LOSS = B // _R_LOSS


def _loss_body(ad_emb, u_row, ts_row, clk_row, ids_row, qp_row,
               u_col, ts_col, clk_col, ids_col,
               w1, b1, w2, b2, w3, b3, out, acc):
    i = pl.program_id(0)

    @pl.when(i == 0)
    def _init():
        acc[0] = 0.0
        acc[1] = 0.0

    ad = ad_emb[...]
    clicked = clk_row[...] == 1
    same_user = u_col[...] == u_row[...]
    causal = ts_col[...] > ts_row[...]
    maskf = (clicked & same_user & causal).astype(_F32)
    msum = jnp.sum(maskf, axis=1, keepdims=True)
    hist = _dot(maskf, ad) / (msum + 1e-16)
    hist_n2 = jnp.sum(hist * hist, axis=1, keepdims=True)
    g = _silu(_dot_t(hist, w1[...]) + b1[...])
    g = _silu(_dot_t(g, w2[...]) + b2[...])
    g = _dot_t(g, w3[...]) + b3[...]
    x = _l2norm(_l2norm(g))
    user_emb = jnp.where(hist_n2 == 0.0, 0.0, x)
    u_n2 = jnp.sum(user_emb * user_emb, axis=1, keepdims=True)
    validf = ((clk_col[...] == 1) & (u_n2 != 0.0)).astype(_F32)
    logits = _dot_t(user_emb, ad) - jnp.log(qp_row[...])
    cols = lax.broadcasted_iota(jnp.int32, (_R_LOSS, B), 1)
    rows = lax.broadcasted_iota(jnp.int32, (_R_LOSS, B), 0) + i * _R_LOSS
    eye = cols == rows
    acc_hits = (ids_col[...] == ids_row[...]) & jnp.logical_not(eye)
    logits = jnp.where(acc_hits, -1e9, logits)
    m = jnp.max(logits, axis=1, keepdims=True)
    lse = m + jnp.log(jnp.sum(jnp.exp(logits - m), axis=1, keepdims=True))
    diag = jnp.sum(jnp.where(eye, logits, 0.0), axis=1, keepdims=True)
    pos_logp = diag - lse
    acc[0] += jnp.sum(pos_logp * validf)
    acc[1] += jnp.sum(validf)

    @pl.when(i == _NB_LOSS - 1)
    def _fin():
        out[...] = jnp.broadcast_to(-acc[0] / (acc[1] + 1e-16), (1, 1))


def _loss(ad_emb, user, timestamp, is_click, ad_ids, q_proba,
          Wu1, bu1, Wu2, bu2, Wu3, bu3):
    full = lambda shape: pl.BlockSpec(shape, lambda i: (0, 0))
    col = pl.BlockSpec((_R_LOSS, 1), lambda i: (i, 0))
    u_row = user.reshape(1, B)
    ts_row = timestamp.reshape(1, B)
    clk_row = is_click.reshape(1, B)
    ids_row = ad_ids.reshape(1, B)
    out = pl.pallas_call(
        _loss_body,
        grid=(_NB_LOSS,),
        in_specs=[
            full((B, D)),
            full((1, B)), full((1, B)), full((1, B)), full((1, B)),
            full((1, B)),
            col, col, col, col,
            full((256, D)), full((1, 256)),
            full((128, 256)), full((1, 128)),
            full((64, 128)), full((1, 64)),
        ],
        out_specs=full((1, 1)),
        out_shape=jax.ShapeDtypeStruct((1, 1), _F32),
        scratch_shapes=[pltpu.SMEM((2,), _F32)],
    )(ad_emb, u_row, ts_row, clk_row, ids_row, q_proba.reshape(1, B),
      u_row.reshape(B, 1), ts_row.reshape(B, 1), clk_row.reshape(B, 1),
      ids_row.reshape(B, 1),
      Wu1, bu1.reshape(1, -1), Wu2, bu2.reshape(1, -1),
      Wu3, bu3.reshape(1, -1))
    return out[0, 0]


def kernel(adgroup_id, cate_id, brand, user, timestamp, is_click, q_proba,
           T_adgroup, T_cate, T_brand, Wa1, ba1, Wa2, ba2, Wa3, ba3,
           Wu1, bu1, Wu2, bu2, Wu3, bu3):
    ia = adgroup_id.reshape(B).astype(jnp.int32)
    ic = cate_id.reshape(B).astype(jnp.int32)
    ib = brand.reshape(B).astype(jnp.int32)
    ea, ec, eb = _sc_gather3(ia, ic, ib, T_adgroup, T_cate, T_brand)
    ad_emb = _ad_mlp(ea, ec, eb, Wa1, ba1, Wa2, ba2, Wa3, ba3)
    return _loss(ad_emb, user.reshape(B), timestamp, is_click.astype(jnp.int32),
                 ia, q_proba, Wu1, bu1, Wu2, bu2, Wu3, bu3)


# speedup vs baseline: 1.2619x; 1.2597x over previous
import functools

import jax
import jax.numpy as jnp
from jax import lax
from jax.experimental import pallas as pl
from jax.experimental.pallas import tpu as pltpu
from jax.experimental.pallas import tpu_sc as plsc

B = 4096
D = 64
_NC = 2
_NS = 16
_NW = _NC * _NS
_BPW = B // _NW

_F32 = jnp.float32


def _dot_t(a, b):
    return lax.dot_general(a, b, (((1,), (1,)), ((), ())),
                           preferred_element_type=_F32)


def _dot(a, b):
    return lax.dot_general(a, b, (((1,), (0,)), ((), ())),
                           preferred_element_type=_F32)


def _silu(x):
    return x / (1.0 + jnp.exp(-x))


def _l2norm(x):
    n = jnp.sqrt(jnp.sum(x * x, axis=-1, keepdims=True))
    return x / jnp.maximum(n, 1e-16)


_HALF = B // 2
_CH = 512
_UNR = 8


def _sc_gather_cate(idx_c, T_c):
    mesh = plsc.ScalarSubcoreMesh(axis_name="c", num_cores=_NC)

    @functools.partial(
        pl.kernel,
        out_type=jax.ShapeDtypeStruct((B, D), _F32),
        mesh=mesh,
        scratch_types=[
            pltpu.SMEM((_CH,), jnp.int32),
            pltpu.SemaphoreType.DMA,
        ],
    )
    def gather_k(ic_h, Tc_h, oc_h, ism, sem):
        cid = lax.axis_index("c")
        base = cid * _HALF
        for ch in range(_HALF // _CH):
            off = base + ch * _CH
            pltpu.sync_copy(ic_h.at[pl.ds(off, _CH)], ism)

            def issue(j, carry, off=off):
                for k in range(_UNR):
                    pltpu.async_copy(
                        Tc_h.at[pl.ds(ism[j * _UNR + k], 1)],
                        oc_h.at[pl.ds(off + j * _UNR + k, 1)], sem)
                return carry

            lax.fori_loop(0, _CH // _UNR, issue, 0)
        pltpu.make_async_copy(Tc_h.at[pl.ds(0, _HALF)],
                              oc_h.at[pl.ds(base, _HALF)], sem).wait()

    return gather_k(idx_c, T_c)


_R_G = 512
_NB_G = B // _R_G


def _tc_gather2_body(ia_s, ib_s, Ta_h, Tb_h, oa, ob, sa, sb):
    g = pl.program_id(0)
    base = g * _R_G

    def issue(j, carry):
        for k in range(_UNR):
            r = j * _UNR + k
            pltpu.async_copy(Ta_h.at[pl.ds(ia_s[base + r], 1)],
                             oa.at[pl.ds(r, 1)], sa)
            pltpu.async_copy(Tb_h.at[pl.ds(ib_s[base + r], 1)],
                             ob.at[pl.ds(r, 1)], sb)
        return carry

    lax.fori_loop(0, _R_G // _UNR, issue, 0)
    pltpu.make_async_copy(Ta_h.at[pl.ds(0, _R_G)], oa, sa).wait()
    pltpu.make_async_copy(Tb_h.at[pl.ds(0, _R_G)], ob, sb).wait()


def _tc_gather2(idx_a, idx_b, T_a, T_b):
    row = pl.BlockSpec((_R_G, D), lambda i: (i, 0))
    smem = pl.BlockSpec(memory_space=pltpu.MemorySpace.SMEM)
    return pl.pallas_call(
        _tc_gather2_body,
        grid=(_NB_G,),
        in_specs=[smem, smem,
                  pl.BlockSpec(memory_space=pltpu.MemorySpace.HBM),
                  pl.BlockSpec(memory_space=pltpu.MemorySpace.HBM)],
        out_specs=[row, row],
        out_shape=[jax.ShapeDtypeStruct((B, D), _F32),
                   jax.ShapeDtypeStruct((B, D), _F32)],
        scratch_shapes=[pltpu.SemaphoreType.DMA, pltpu.SemaphoreType.DMA],
    )(idx_a, idx_b, T_a, T_b)


_R_MLP = 512


def _ad_mlp_body(ea, ec, eb, w1a, w1c, w1b, b1, w2, b2, w3, b3, out):
    h = _dot_t(ea[...], w1a[...]) + _dot_t(ec[...], w1c[...]) \
        + _dot_t(eb[...], w1b[...]) + b1[...]
    h = _silu(h)
    h = _silu(_dot_t(h, w2[...]) + b2[...])
    h = _dot_t(h, w3[...]) + b3[...]
    out[...] = _l2norm(_l2norm(h))


def _ad_mlp(ea, ec, eb, Wa1, ba1, Wa2, ba2, Wa3, ba3):
    full = lambda shape: pl.BlockSpec(shape, lambda i: (0, 0))
    row = pl.BlockSpec((_R_MLP, D), lambda i: (i, 0))
    return pl.pallas_call(
        _ad_mlp_body,
        grid=(B // _R_MLP,),
        in_specs=[
            row, row, row,
            full((256, D)), full((256, D)), full((256, D)), full((1, 256)),
            full((128, 256)), full((1, 128)),
            full((64, 128)), full((1, 64)),
        ],
        out_specs=row,
        out_shape=jax.ShapeDtypeStruct((B, D), _F32),
    )(ea, ec, eb, Wa1[:, 0:D], Wa1[:, D:2 * D], Wa1[:, 2 * D:3 * D],
      ba1.reshape(1, -1), Wa2, ba2.reshape(1, -1), Wa3, ba3.reshape(1, -1))


_R_LOSS = 256
_NB_LOSS = B // _R_LOSS


def _loss_body(ad_emb, u_row, ts_row, clk_row, ids_row, qp_row,
               u_col, ts_col, clk_col, ids_col,
               w1, b1, w2, b2, w3, b3, out, acc):
    i = pl.program_id(0)

    @pl.when(i == 0)
    def _init():
        acc[0] = 0.0
        acc[1] = 0.0

    ad = ad_emb[...]
    clicked = clk_row[...] == 1
    same_user = u_col[...] == u_row[...]
    causal = ts_col[...] > ts_row[...]
    maskf = (clicked & same_user & causal).astype(_F32)
    msum = jnp.sum(maskf, axis=1, keepdims=True)
    hist = _dot(maskf, ad) / (msum + 1e-16)
    hist_n2 = jnp.sum(hist * hist, axis=1, keepdims=True)
    g = _silu(_dot_t(hist, w1[...]) + b1[...])
    g = _silu(_dot_t(g, w2[...]) + b2[...])
    g = _dot_t(g, w3[...]) + b3[...]
    x = _l2norm(_l2norm(g))
    user_emb = jnp.where(hist_n2 == 0.0, 0.0, x)
    u_n2 = jnp.sum(user_emb * user_emb, axis=1, keepdims=True)
    validf = ((clk_col[...] == 1) & (u_n2 != 0.0)).astype(_F32)
    logits = _dot_t(user_emb, ad) - jnp.log(qp_row[...])
    cols = lax.broadcasted_iota(jnp.int32, (_R_LOSS, B), 1)
    rows = lax.broadcasted_iota(jnp.int32, (_R_LOSS, B), 0) + i * _R_LOSS
    eye = cols == rows
    acc_hits = (ids_col[...] == ids_row[...]) & jnp.logical_not(eye)
    logits = jnp.where(acc_hits, -1e9, logits)
    m = jnp.max(logits, axis=1, keepdims=True)
    lse = m + jnp.log(jnp.sum(jnp.exp(logits - m), axis=1, keepdims=True))
    diag = jnp.sum(jnp.where(eye, logits, 0.0), axis=1, keepdims=True)
    pos_logp = diag - lse
    acc[0] += jnp.sum(pos_logp * validf)
    acc[1] += jnp.sum(validf)

    @pl.when(i == _NB_LOSS - 1)
    def _fin():
        out[...] = jnp.broadcast_to(-acc[0] / (acc[1] + 1e-16), (1, 1))


def _loss(ad_emb, user, timestamp, is_click, ad_ids, q_proba,
          Wu1, bu1, Wu2, bu2, Wu3, bu3):
    full = lambda shape: pl.BlockSpec(shape, lambda i: (0, 0))
    col = pl.BlockSpec((_R_LOSS, 1), lambda i: (i, 0))
    u_row = user.reshape(1, B)
    ts_row = timestamp.reshape(1, B)
    clk_row = is_click.reshape(1, B)
    ids_row = ad_ids.reshape(1, B)
    out = pl.pallas_call(
        _loss_body,
        grid=(_NB_LOSS,),
        in_specs=[
            full((B, D)),
            full((1, B)), full((1, B)), full((1, B)), full((1, B)),
            full((1, B)),
            col, col, col, col,
            full((256, D)), full((1, 256)),
            full((128, 256)), full((1, 128)),
            full((64, 128)), full((1, 64)),
        ],
        out_specs=full((1, 1)),
        out_shape=jax.ShapeDtypeStruct((1, 1), _F32),
        scratch_shapes=[pltpu.SMEM((2,), _F32)],
    )(ad_emb, u_row, ts_row, clk_row, ids_row, q_proba.reshape(1, B),
      u_row.reshape(B, 1), ts_row.reshape(B, 1), clk_row.reshape(B, 1),
      ids_row.reshape(B, 1),
      Wu1, bu1.reshape(1, -1), Wu2, bu2.reshape(1, -1),
      Wu3, bu3.reshape(1, -1))
    return out[0, 0]


def kernel(adgroup_id, cate_id, brand, user, timestamp, is_click, q_proba,
           T_adgroup, T_cate, T_brand, Wa1, ba1, Wa2, ba2, Wa3, ba3,
           Wu1, bu1, Wu2, bu2, Wu3, bu3):
    ia = adgroup_id.reshape(B).astype(jnp.int32)
    ic = cate_id.reshape(B).astype(jnp.int32)
    ib = brand.reshape(B).astype(jnp.int32)
    ec = _sc_gather_cate(ic, T_cate)
    ea, eb = _tc_gather2(ia, ib, T_adgroup, T_brand)
    ad_emb = _ad_mlp(ea, ec, eb, Wa1, ba1, Wa2, ba2, Wa3, ba3)
    return _loss(ad_emb, user.reshape(B), timestamp, is_click.astype(jnp.int32),
                 ia, q_proba, Wu1, bu1, Wu2, bu2, Wu3, bu3)


# speedup vs baseline: 2.1643x; 1.7151x over previous
import functools

import jax
import jax.numpy as jnp
from jax import lax
from jax.experimental import pallas as pl
from jax.experimental.pallas import tpu as pltpu
from jax.experimental.pallas import tpu_sc as plsc

B = 4096
D = 64
_NC = 2
_NS = 16
_NW = _NC * _NS
_BPW = B // _NW

_F32 = jnp.float32


def _dot_t(a, b):
    return lax.dot_general(a, b, (((1,), (1,)), ((), ())),
                           preferred_element_type=_F32)


def _dot(a, b):
    return lax.dot_general(a, b, (((1,), (0,)), ((), ())),
                           preferred_element_type=_F32)


def _silu(x):
    return x / (1.0 + jnp.exp(-x))


def _l2norm(x):
    n = jnp.sqrt(jnp.sum(x * x, axis=-1, keepdims=True))
    return x / jnp.maximum(n, 1e-16)


_HALF = B // 2
_CH = 512
_UNR = 8


def _sc_gather_cate(idx_c, T_c):
    mesh = plsc.ScalarSubcoreMesh(axis_name="c", num_cores=_NC)

    @functools.partial(
        pl.kernel,
        out_type=jax.ShapeDtypeStruct((B, D), _F32),
        mesh=mesh,
        scratch_types=[
            pltpu.SMEM((_CH,), jnp.int32),
            pltpu.SemaphoreType.DMA,
        ],
    )
    def gather_k(ic_h, Tc_h, oc_h, ism, sem):
        cid = lax.axis_index("c")
        base = cid * _HALF
        for ch in range(_HALF // _CH):
            off = base + ch * _CH
            pltpu.sync_copy(ic_h.at[pl.ds(off, _CH)], ism)

            def issue(j, carry, off=off):
                for k in range(_UNR):
                    pltpu.async_copy(
                        Tc_h.at[pl.ds(ism[j * _UNR + k], 1)],
                        oc_h.at[pl.ds(off + j * _UNR + k, 1)], sem)
                return carry

            lax.fori_loop(0, _CH // _UNR, issue, 0)
        pltpu.make_async_copy(Tc_h.at[pl.ds(0, _HALF)],
                              oc_h.at[pl.ds(base, _HALF)], sem).wait()

    return gather_k(idx_c, T_c)


_R_G = 256
_NB_G = B // _R_G
_LANES = 128


def _tc_gather2_body(ia_s, ib_s, ia_v, ib_v, Ta_h, Tb_h, oa, ob,
                     bufa, bufb, sa, sb):
    g = pl.program_id(0)
    base = g * _R_G

    def issue(j, carry):
        for k in range(_UNR):
            r = j * _UNR + k
            ba_ = pl.multiple_of((ia_s[base + r] >> 7) * _LANES, _LANES)
            bb_ = pl.multiple_of((ib_s[base + r] >> 7) * _LANES, _LANES)
            pltpu.async_copy(Ta_h.at[:, pl.ds(ba_, _LANES)], bufa.at[r], sa)
            pltpu.async_copy(Tb_h.at[:, pl.ds(bb_, _LANES)], bufb.at[r], sb)
        return carry

    lax.fori_loop(0, _R_G // _UNR, issue, 0)

    def drain(j, carry):
        pltpu.make_async_copy(Ta_h.at[:, pl.ds(0, _LANES)], bufa.at[0], sa).wait()
        pltpu.make_async_copy(Tb_h.at[:, pl.ds(0, _LANES)], bufb.at[0], sb).wait()
        return carry

    lax.fori_loop(0, _R_G, drain, 0)
    lane = lax.broadcasted_iota(jnp.int32, (_R_G, 1, _LANES), 2)
    ra = (ia_v[...] & (_LANES - 1)).reshape(_R_G, 1, 1)
    rb = (ib_v[...] & (_LANES - 1)).reshape(_R_G, 1, 1)
    oa[...] = jnp.sum(jnp.where(lane == ra, bufa[...], 0.0), axis=2)
    ob[...] = jnp.sum(jnp.where(lane == rb, bufb[...], 0.0), axis=2)


def _tc_gather2(idx_a, idx_b, T_aT, T_bT):
    row = pl.BlockSpec((_R_G, D), lambda i: (i, 0))
    col1 = pl.BlockSpec((_R_G, 1), lambda i: (i, 0))
    smem = pl.BlockSpec(memory_space=pltpu.MemorySpace.SMEM)
    return pl.pallas_call(
        _tc_gather2_body,
        grid=(_NB_G,),
        in_specs=[smem, smem, col1, col1,
                  pl.BlockSpec(memory_space=pltpu.MemorySpace.HBM),
                  pl.BlockSpec(memory_space=pltpu.MemorySpace.HBM)],
        out_specs=[row, row],
        out_shape=[jax.ShapeDtypeStruct((B, D), _F32),
                   jax.ShapeDtypeStruct((B, D), _F32)],
        scratch_shapes=[pltpu.VMEM((_R_G, D, _LANES), _F32),
                        pltpu.VMEM((_R_G, D, _LANES), _F32),
                        pltpu.SemaphoreType.DMA, pltpu.SemaphoreType.DMA],
    )(idx_a, idx_b, idx_a.reshape(B, 1), idx_b.reshape(B, 1), T_aT, T_bT)


_R_MLP = 512


def _ad_mlp_body(ea, ec, eb, w1a, w1c, w1b, b1, w2, b2, w3, b3, out):
    h = _dot_t(ea[...], w1a[...]) + _dot_t(ec[...], w1c[...]) \
        + _dot_t(eb[...], w1b[...]) + b1[...]
    h = _silu(h)
    h = _silu(_dot_t(h, w2[...]) + b2[...])
    h = _dot_t(h, w3[...]) + b3[...]
    out[...] = _l2norm(_l2norm(h))


def _ad_mlp(ea, ec, eb, Wa1, ba1, Wa2, ba2, Wa3, ba3):
    full = lambda shape: pl.BlockSpec(shape, lambda i: (0, 0))
    row = pl.BlockSpec((_R_MLP, D), lambda i: (i, 0))
    return pl.pallas_call(
        _ad_mlp_body,
        grid=(B // _R_MLP,),
        in_specs=[
            row, row, row,
            full((256, D)), full((256, D)), full((256, D)), full((1, 256)),
            full((128, 256)), full((1, 128)),
            full((64, 128)), full((1, 64)),
        ],
        out_specs=row,
        out_shape=jax.ShapeDtypeStruct((B, D), _F32),
    )(ea, ec, eb, Wa1[:, 0:D], Wa1[:, D:2 * D], Wa1[:, 2 * D:3 * D],
      ba1.reshape(1, -1), Wa2, ba2.reshape(1, -1), Wa3, ba3.reshape(1, -1))


_R_LOSS = 256
_NB_LOSS = B // _R_LOSS


def _loss_body(ad_emb, u_row, ts_row, clk_row, ids_row, qp_row,
               u_col, ts_col, clk_col, ids_col,
               w1, b1, w2, b2, w3, b3, out, acc):
    i = pl.program_id(0)

    @pl.when(i == 0)
    def _init():
        acc[0] = 0.0
        acc[1] = 0.0

    ad = ad_emb[...]
    clicked = clk_row[...] == 1
    same_user = u_col[...] == u_row[...]
    causal = ts_col[...] > ts_row[...]
    maskf = (clicked & same_user & causal).astype(_F32)
    msum = jnp.sum(maskf, axis=1, keepdims=True)
    hist = _dot(maskf, ad) / (msum + 1e-16)
    hist_n2 = jnp.sum(hist * hist, axis=1, keepdims=True)
    g = _silu(_dot_t(hist, w1[...]) + b1[...])
    g = _silu(_dot_t(g, w2[...]) + b2[...])
    g = _dot_t(g, w3[...]) + b3[...]
    x = _l2norm(_l2norm(g))
    user_emb = jnp.where(hist_n2 == 0.0, 0.0, x)
    u_n2 = jnp.sum(user_emb * user_emb, axis=1, keepdims=True)
    validf = ((clk_col[...] == 1) & (u_n2 != 0.0)).astype(_F32)
    logits = _dot_t(user_emb, ad) - jnp.log(qp_row[...])
    cols = lax.broadcasted_iota(jnp.int32, (_R_LOSS, B), 1)
    rows = lax.broadcasted_iota(jnp.int32, (_R_LOSS, B), 0) + i * _R_LOSS
    eye = cols == rows
    acc_hits = (ids_col[...] == ids_row[...]) & jnp.logical_not(eye)
    logits = jnp.where(acc_hits, -1e9, logits)
    m = jnp.max(logits, axis=1, keepdims=True)
    lse = m + jnp.log(jnp.sum(jnp.exp(logits - m), axis=1, keepdims=True))
    diag = jnp.sum(jnp.where(eye, logits, 0.0), axis=1, keepdims=True)
    pos_logp = diag - lse
    acc[0] += jnp.sum(pos_logp * validf)
    acc[1] += jnp.sum(validf)

    @pl.when(i == _NB_LOSS - 1)
    def _fin():
        out[...] = jnp.broadcast_to(-acc[0] / (acc[1] + 1e-16), (1, 1))


def _loss(ad_emb, user, timestamp, is_click, ad_ids, q_proba,
          Wu1, bu1, Wu2, bu2, Wu3, bu3):
    full = lambda shape: pl.BlockSpec(shape, lambda i: (0, 0))
    col = pl.BlockSpec((_R_LOSS, 1), lambda i: (i, 0))
    u_row = user.reshape(1, B)
    ts_row = timestamp.reshape(1, B)
    clk_row = is_click.reshape(1, B)
    ids_row = ad_ids.reshape(1, B)
    out = pl.pallas_call(
        _loss_body,
        grid=(_NB_LOSS,),
        in_specs=[
            full((B, D)),
            full((1, B)), full((1, B)), full((1, B)), full((1, B)),
            full((1, B)),
            col, col, col, col,
            full((256, D)), full((1, 256)),
            full((128, 256)), full((1, 128)),
            full((64, 128)), full((1, 64)),
        ],
        out_specs=full((1, 1)),
        out_shape=jax.ShapeDtypeStruct((1, 1), _F32),
        scratch_shapes=[pltpu.SMEM((2,), _F32)],
    )(ad_emb, u_row, ts_row, clk_row, ids_row, q_proba.reshape(1, B),
      u_row.reshape(B, 1), ts_row.reshape(B, 1), clk_row.reshape(B, 1),
      ids_row.reshape(B, 1),
      Wu1, bu1.reshape(1, -1), Wu2, bu2.reshape(1, -1),
      Wu3, bu3.reshape(1, -1))
    return out[0, 0]


def kernel(adgroup_id, cate_id, brand, user, timestamp, is_click, q_proba,
           T_adgroup, T_cate, T_brand, Wa1, ba1, Wa2, ba2, Wa3, ba3,
           Wu1, bu1, Wu2, bu2, Wu3, bu3):
    ia = adgroup_id.reshape(B).astype(jnp.int32)
    ic = cate_id.reshape(B).astype(jnp.int32)
    ib = brand.reshape(B).astype(jnp.int32)
    ec = _sc_gather_cate(ic, T_cate)
    ea, eb = _tc_gather2(ia, ib, T_adgroup.T, T_brand.T)
    ad_emb = _ad_mlp(ea, ec, eb, Wa1, ba1, Wa2, ba2, Wa3, ba3)
    return _loss(ad_emb, user.reshape(B), timestamp, is_click.astype(jnp.int32),
                 ia, q_proba, Wu1, bu1, Wu2, bu2, Wu3, bu3)


# speedup vs baseline: 2.1686x; 1.0020x over previous
import functools

import jax
import jax.numpy as jnp
from jax import lax
from jax.experimental import pallas as pl
from jax.experimental.pallas import tpu as pltpu
from jax.experimental.pallas import tpu_sc as plsc

B = 4096
D = 64
_NC = 2
_NS = 16
_NW = _NC * _NS
_BPW = B // _NW

_F32 = jnp.float32


def _dot_t(a, b):
    return lax.dot_general(a, b, (((1,), (1,)), ((), ())),
                           preferred_element_type=_F32)


def _dot(a, b):
    return lax.dot_general(a, b, (((1,), (0,)), ((), ())),
                           preferred_element_type=_F32)


def _silu(x):
    return x / (1.0 + jnp.exp(-x))


def _l2norm(x):
    n = jnp.sqrt(jnp.sum(x * x, axis=-1, keepdims=True))
    return x / jnp.maximum(n, 1e-16)


_HALF = B // 2
_CH = 512
_UNR = 8


def _sc_gather_cate(idx_c, T_c):
    mesh = plsc.ScalarSubcoreMesh(axis_name="c", num_cores=_NC)

    @functools.partial(
        pl.kernel,
        out_type=jax.ShapeDtypeStruct((B, D), _F32),
        mesh=mesh,
        scratch_types=[
            pltpu.SMEM((_CH,), jnp.int32),
            pltpu.SemaphoreType.DMA,
        ],
    )
    def gather_k(ic_h, Tc_h, oc_h, ism, sem):
        cid = lax.axis_index("c")
        base = cid * _HALF
        for ch in range(_HALF // _CH):
            off = base + ch * _CH
            pltpu.sync_copy(ic_h.at[pl.ds(off, _CH)], ism)

            def issue(j, carry, off=off):
                for k in range(_UNR):
                    pltpu.async_copy(
                        Tc_h.at[pl.ds(ism[j * _UNR + k], 1)],
                        oc_h.at[pl.ds(off + j * _UNR + k, 1)], sem)
                return carry

            lax.fori_loop(0, _CH // _UNR, issue, 0)
        pltpu.make_async_copy(Tc_h.at[pl.ds(0, _HALF)],
                              oc_h.at[pl.ds(base, _HALF)], sem).wait()

    return gather_k(idx_c, T_c)


_R_G = 256
_NB_G = B // _R_G
_LANES = 128


def _tc_gather2_body(ia_s, ib_s, ia_v, ib_v, Ta_h, Tb_h, oa, ob,
                     bufa, bufb, sa, sb):
    g = pl.program_id(0)
    base = g * _R_G

    def issue(j, carry):
        for k in range(_UNR):
            r = j * _UNR + k
            ba_ = pl.multiple_of((ia_s[base + r] >> 7) * _LANES, _LANES)
            bb_ = pl.multiple_of((ib_s[base + r] >> 7) * _LANES, _LANES)
            pltpu.async_copy(Ta_h.at[:, pl.ds(ba_, _LANES)], bufa.at[r], sa)
            pltpu.async_copy(Tb_h.at[:, pl.ds(bb_, _LANES)], bufb.at[r], sb)
        return carry

    lax.fori_loop(0, _R_G // _UNR, issue, 0)

    pltpu.make_async_copy(bufa, bufa, sa).wait()
    pltpu.make_async_copy(bufb, bufb, sb).wait()
    lane = lax.broadcasted_iota(jnp.int32, (_R_G, 1, _LANES), 2)
    ra = (ia_v[...] & (_LANES - 1)).reshape(_R_G, 1, 1)
    rb = (ib_v[...] & (_LANES - 1)).reshape(_R_G, 1, 1)
    oa[...] = jnp.sum(jnp.where(lane == ra, bufa[...], 0.0), axis=2)
    ob[...] = jnp.sum(jnp.where(lane == rb, bufb[...], 0.0), axis=2)


def _tc_gather2(idx_a, idx_b, T_aT, T_bT):
    row = pl.BlockSpec((_R_G, D), lambda i: (i, 0))
    col1 = pl.BlockSpec((_R_G, 1), lambda i: (i, 0))
    smem = pl.BlockSpec(memory_space=pltpu.MemorySpace.SMEM)
    return pl.pallas_call(
        _tc_gather2_body,
        grid=(_NB_G,),
        in_specs=[smem, smem, col1, col1,
                  pl.BlockSpec(memory_space=pltpu.MemorySpace.HBM),
                  pl.BlockSpec(memory_space=pltpu.MemorySpace.HBM)],
        out_specs=[row, row],
        out_shape=[jax.ShapeDtypeStruct((B, D), _F32),
                   jax.ShapeDtypeStruct((B, D), _F32)],
        scratch_shapes=[pltpu.VMEM((_R_G, D, _LANES), _F32),
                        pltpu.VMEM((_R_G, D, _LANES), _F32),
                        pltpu.SemaphoreType.DMA, pltpu.SemaphoreType.DMA],
    )(idx_a, idx_b, idx_a.reshape(B, 1), idx_b.reshape(B, 1), T_aT, T_bT)


_R_MLP = 512


def _ad_mlp_body(ea, ec, eb, w1a, w1c, w1b, b1, w2, b2, w3, b3, out):
    h = _dot_t(ea[...], w1a[...]) + _dot_t(ec[...], w1c[...]) \
        + _dot_t(eb[...], w1b[...]) + b1[...]
    h = _silu(h)
    h = _silu(_dot_t(h, w2[...]) + b2[...])
    h = _dot_t(h, w3[...]) + b3[...]
    out[...] = _l2norm(_l2norm(h))


def _ad_mlp(ea, ec, eb, Wa1, ba1, Wa2, ba2, Wa3, ba3):
    full = lambda shape: pl.BlockSpec(shape, lambda i: (0, 0))
    row = pl.BlockSpec((_R_MLP, D), lambda i: (i, 0))
    return pl.pallas_call(
        _ad_mlp_body,
        grid=(B // _R_MLP,),
        in_specs=[
            row, row, row,
            full((256, D)), full((256, D)), full((256, D)), full((1, 256)),
            full((128, 256)), full((1, 128)),
            full((64, 128)), full((1, 64)),
        ],
        out_specs=row,
        out_shape=jax.ShapeDtypeStruct((B, D), _F32),
    )(ea, ec, eb, Wa1[:, 0:D], Wa1[:, D:2 * D], Wa1[:, 2 * D:3 * D],
      ba1.reshape(1, -1), Wa2, ba2.reshape(1, -1), Wa3, ba3.reshape(1, -1))


_R_LOSS = 256
_NB_LOSS = B // _R_LOSS


def _loss_body(ad_emb, u_row, ts_row, clk_row, ids_row, qp_row,
               u_col, ts_col, clk_col, ids_col,
               w1, b1, w2, b2, w3, b3, out, acc):
    i = pl.program_id(0)

    @pl.when(i == 0)
    def _init():
        acc[0] = 0.0
        acc[1] = 0.0

    ad = ad_emb[...]
    clicked = clk_row[...] == 1
    same_user = u_col[...] == u_row[...]
    causal = ts_col[...] > ts_row[...]
    maskf = (clicked & same_user & causal).astype(_F32)
    msum = jnp.sum(maskf, axis=1, keepdims=True)
    hist = _dot(maskf, ad) / (msum + 1e-16)
    hist_n2 = jnp.sum(hist * hist, axis=1, keepdims=True)
    g = _silu(_dot_t(hist, w1[...]) + b1[...])
    g = _silu(_dot_t(g, w2[...]) + b2[...])
    g = _dot_t(g, w3[...]) + b3[...]
    x = _l2norm(_l2norm(g))
    user_emb = jnp.where(hist_n2 == 0.0, 0.0, x)
    u_n2 = jnp.sum(user_emb * user_emb, axis=1, keepdims=True)
    validf = ((clk_col[...] == 1) & (u_n2 != 0.0)).astype(_F32)
    logits = _dot_t(user_emb, ad) - jnp.log(qp_row[...])
    cols = lax.broadcasted_iota(jnp.int32, (_R_LOSS, B), 1)
    rows = lax.broadcasted_iota(jnp.int32, (_R_LOSS, B), 0) + i * _R_LOSS
    eye = cols == rows
    acc_hits = (ids_col[...] == ids_row[...]) & jnp.logical_not(eye)
    logits = jnp.where(acc_hits, -1e9, logits)
    m = jnp.max(logits, axis=1, keepdims=True)
    lse = m + jnp.log(jnp.sum(jnp.exp(logits - m), axis=1, keepdims=True))
    diag = jnp.sum(jnp.where(eye, logits, 0.0), axis=1, keepdims=True)
    pos_logp = diag - lse
    acc[0] += jnp.sum(pos_logp * validf)
    acc[1] += jnp.sum(validf)

    @pl.when(i == _NB_LOSS - 1)
    def _fin():
        out[...] = jnp.broadcast_to(-acc[0] / (acc[1] + 1e-16), (1, 1))


def _loss(ad_emb, user, timestamp, is_click, ad_ids, q_proba,
          Wu1, bu1, Wu2, bu2, Wu3, bu3):
    full = lambda shape: pl.BlockSpec(shape, lambda i: (0, 0))
    col = pl.BlockSpec((_R_LOSS, 1), lambda i: (i, 0))
    u_row = user.reshape(1, B)
    ts_row = timestamp.reshape(1, B)
    clk_row = is_click.reshape(1, B)
    ids_row = ad_ids.reshape(1, B)
    out = pl.pallas_call(
        _loss_body,
        grid=(_NB_LOSS,),
        in_specs=[
            full((B, D)),
            full((1, B)), full((1, B)), full((1, B)), full((1, B)),
            full((1, B)),
            col, col, col, col,
            full((256, D)), full((1, 256)),
            full((128, 256)), full((1, 128)),
            full((64, 128)), full((1, 64)),
        ],
        out_specs=full((1, 1)),
        out_shape=jax.ShapeDtypeStruct((1, 1), _F32),
        scratch_shapes=[pltpu.SMEM((2,), _F32)],
    )(ad_emb, u_row, ts_row, clk_row, ids_row, q_proba.reshape(1, B),
      u_row.reshape(B, 1), ts_row.reshape(B, 1), clk_row.reshape(B, 1),
      ids_row.reshape(B, 1),
      Wu1, bu1.reshape(1, -1), Wu2, bu2.reshape(1, -1),
      Wu3, bu3.reshape(1, -1))
    return out[0, 0]


def kernel(adgroup_id, cate_id, brand, user, timestamp, is_click, q_proba,
           T_adgroup, T_cate, T_brand, Wa1, ba1, Wa2, ba2, Wa3, ba3,
           Wu1, bu1, Wu2, bu2, Wu3, bu3):
    ia = adgroup_id.reshape(B).astype(jnp.int32)
    ic = cate_id.reshape(B).astype(jnp.int32)
    ib = brand.reshape(B).astype(jnp.int32)
    ec = _sc_gather_cate(ic, T_cate)
    ea, eb = _tc_gather2(ia, ib, T_adgroup.T, T_brand.T)
    ad_emb = _ad_mlp(ea, ec, eb, Wa1, ba1, Wa2, ba2, Wa3, ba3)
    return _loss(ad_emb, user.reshape(B), timestamp, is_click.astype(jnp.int32),
                 ia, q_proba, Wu1, bu1, Wu2, bu2, Wu3, bu3)


# speedup vs baseline: 2.2124x; 1.0202x over previous
import functools

import jax
import jax.numpy as jnp
from jax import lax
from jax.experimental import pallas as pl
from jax.experimental.pallas import tpu as pltpu
from jax.experimental.pallas import tpu_sc as plsc

B = 4096
D = 64
_NC = 2
_NS = 16
_NW = _NC * _NS
_BPW = B // _NW

_F32 = jnp.float32


def _dot_t(a, b):
    return lax.dot_general(a, b, (((1,), (1,)), ((), ())),
                           preferred_element_type=_F32)


def _dot(a, b):
    return lax.dot_general(a, b, (((1,), (0,)), ((), ())),
                           preferred_element_type=_F32)


def _silu(x):
    return x / (1.0 + jnp.exp(-x))


def _l2norm(x):
    n = jnp.sqrt(jnp.sum(x * x, axis=-1, keepdims=True))
    return x / jnp.maximum(n, 1e-16)


_HALF = B // 2
_CH = 512
_UNR = 8


def _sc_gather_cate(idx_c, T_c):
    mesh = plsc.ScalarSubcoreMesh(axis_name="c", num_cores=_NC)

    @functools.partial(
        pl.kernel,
        out_type=jax.ShapeDtypeStruct((B, D), _F32),
        mesh=mesh,
        scratch_types=[
            pltpu.SMEM((_CH,), jnp.int32),
            pltpu.SemaphoreType.DMA,
        ],
    )
    def gather_k(ic_h, Tc_h, oc_h, ism, sem):
        cid = lax.axis_index("c")
        base = cid * _HALF
        for ch in range(_HALF // _CH):
            off = base + ch * _CH
            pltpu.sync_copy(ic_h.at[pl.ds(off, _CH)], ism)

            def issue(j, carry, off=off):
                for k in range(_UNR):
                    pltpu.async_copy(
                        Tc_h.at[pl.ds(ism[j * _UNR + k], 1)],
                        oc_h.at[pl.ds(off + j * _UNR + k, 1)], sem)
                return carry

            lax.fori_loop(0, _CH // _UNR, issue, 0)
        pltpu.make_async_copy(Tc_h.at[pl.ds(0, _HALF)],
                              oc_h.at[pl.ds(base, _HALF)], sem).wait()

    return gather_k(idx_c, T_c)


_R_G = 256
_NB_G = B // _R_G
_LANES = 128


def _tc_gather2_body(ia_s, ib_s, ia_v, ib_v, Ta_h, Tb_h, oa, ob,
                     bufa, bufb, sa, sb):
    g = pl.program_id(0)
    base = g * _R_G

    def issue(j, carry):
        for k in range(_UNR):
            r = j * _UNR + k
            ba_ = pl.multiple_of((ia_s[base + r] >> 7) * _LANES, _LANES)
            bb_ = pl.multiple_of((ib_s[base + r] >> 7) * _LANES, _LANES)
            pltpu.async_copy(Ta_h.at[:, pl.ds(ba_, _LANES)], bufa.at[r], sa)
            pltpu.async_copy(Tb_h.at[:, pl.ds(bb_, _LANES)], bufb.at[r], sb)
        return carry

    lax.fori_loop(0, _R_G // _UNR, issue, 0)

    pltpu.make_async_copy(bufa, bufa, sa).wait()
    pltpu.make_async_copy(bufb, bufb, sb).wait()
    lane = lax.broadcasted_iota(jnp.int32, (_R_G, 1, _LANES), 2)
    ra = (ia_v[...] & (_LANES - 1)).reshape(_R_G, 1, 1)
    rb = (ib_v[...] & (_LANES - 1)).reshape(_R_G, 1, 1)
    oa[...] = jnp.sum(jnp.where(lane == ra, bufa[...], 0.0), axis=2)
    ob[...] = jnp.sum(jnp.where(lane == rb, bufb[...], 0.0), axis=2)


def _tc_gather2(idx_a, idx_b, T_aT, T_bT):
    row = pl.BlockSpec((_R_G, D), lambda i: (i, 0))
    col1 = pl.BlockSpec((_R_G, 1), lambda i: (i, 0))
    smem = pl.BlockSpec(memory_space=pltpu.MemorySpace.SMEM)
    return pl.pallas_call(
        _tc_gather2_body,
        grid=(_NB_G,),
        in_specs=[smem, smem, col1, col1,
                  pl.BlockSpec(memory_space=pltpu.MemorySpace.HBM),
                  pl.BlockSpec(memory_space=pltpu.MemorySpace.HBM)],
        out_specs=[row, row],
        out_shape=[jax.ShapeDtypeStruct((B, D), _F32),
                   jax.ShapeDtypeStruct((B, D), _F32)],
        scratch_shapes=[pltpu.VMEM((_R_G, D, _LANES), _F32),
                        pltpu.VMEM((_R_G, D, _LANES), _F32),
                        pltpu.SemaphoreType.DMA, pltpu.SemaphoreType.DMA],
    )(idx_a, idx_b, idx_a.reshape(B, 1), idx_b.reshape(B, 1), T_aT, T_bT)


_R_MLP = 512


def _ad_mlp_body(ea, ec, eb, w1a, w1c, w1b, b1, w2, b2, w3, b3, out):
    h = _dot_t(ea[...], w1a[...]) + _dot_t(ec[...], w1c[...]) \
        + _dot_t(eb[...], w1b[...]) + b1[...]
    h = _silu(h)
    h = _silu(_dot_t(h, w2[...]) + b2[...])
    h = _dot_t(h, w3[...]) + b3[...]
    out[...] = _l2norm(_l2norm(h))


def _ad_mlp(ea, ec, eb, Wa1, ba1, Wa2, ba2, Wa3, ba3):
    full = lambda shape: pl.BlockSpec(shape, lambda i: (0, 0))
    row = pl.BlockSpec((_R_MLP, D), lambda i: (i, 0))
    return pl.pallas_call(
        _ad_mlp_body,
        grid=(B // _R_MLP,),
        in_specs=[
            row, row, row,
            full((256, D)), full((256, D)), full((256, D)), full((1, 256)),
            full((128, 256)), full((1, 128)),
            full((64, 128)), full((1, 64)),
        ],
        out_specs=row,
        out_shape=jax.ShapeDtypeStruct((B, D), _F32),
    )(ea, ec, eb, Wa1[:, 0:D], Wa1[:, D:2 * D], Wa1[:, 2 * D:3 * D],
      ba1.reshape(1, -1), Wa2, ba2.reshape(1, -1), Wa3, ba3.reshape(1, -1))


_R_LOSS = 256
_NB_LOSS = B // _R_LOSS


def _loss_body(ad_emb, u_row, ts_row, clk_row, ids_row, qp_row,
               u_col, ts_col, clk_col, ids_col, qp_col,
               w1, b1, w2, b2, w3, b3, out, acc):
    i = pl.program_id(0)

    @pl.when(i == 0)
    def _init():
        acc[0] = 0.0
        acc[1] = 0.0

    ad = ad_emb[...]
    krow = 2 * u_row[...] + clk_row[...]
    kcol = 2 * u_col[...] + 1
    maskf = ((kcol == krow) & (ts_col[...] > ts_row[...])).astype(_F32)
    msum = jnp.sum(maskf, axis=1, keepdims=True)
    hist = _dot(maskf, ad) / (msum + 1e-16)
    hist_n2 = jnp.sum(hist * hist, axis=1, keepdims=True)
    g = _silu(_dot_t(hist, w1[...]) + b1[...])
    g = _silu(_dot_t(g, w2[...]) + b2[...])
    g = _dot_t(g, w3[...]) + b3[...]
    x = _l2norm(_l2norm(g))
    user_emb = jnp.where(hist_n2 == 0.0, 0.0, x)
    u_n2 = jnp.sum(user_emb * user_emb, axis=1, keepdims=True)
    validf = ((clk_col[...] == 1) & (u_n2 != 0.0)).astype(_F32)
    logits = _dot_t(user_emb, ad) - jnp.log(qp_row[...])
    cols = lax.broadcasted_iota(jnp.int32, (_R_LOSS, B), 1)
    rows = lax.broadcasted_iota(jnp.int32, (_R_LOSS, B), 0) + i * _R_LOSS
    eye = cols == rows
    acc_hits = (ids_col[...] == ids_row[...]) != eye
    logits = jnp.where(acc_hits, -1e9, logits)
    m = jnp.max(logits, axis=1, keepdims=True)
    lse = m + jnp.log(jnp.sum(jnp.exp(logits - m), axis=1, keepdims=True))
    ad_blk = ad_emb[pl.ds(i * _R_LOSS, _R_LOSS), :]
    diag = jnp.sum(user_emb * ad_blk, axis=1, keepdims=True) \
        - jnp.log(qp_col[...])
    pos_logp = diag - lse
    acc[0] += jnp.sum(pos_logp * validf)
    acc[1] += jnp.sum(validf)

    @pl.when(i == _NB_LOSS - 1)
    def _fin():
        out[...] = jnp.broadcast_to(-acc[0] / (acc[1] + 1e-16), (1, 1))


def _loss(ad_emb, user, timestamp, is_click, ad_ids, q_proba,
          Wu1, bu1, Wu2, bu2, Wu3, bu3):
    full = lambda shape: pl.BlockSpec(shape, lambda i: (0, 0))
    col = pl.BlockSpec((_R_LOSS, 1), lambda i: (i, 0))
    u_row = user.reshape(1, B)
    ts_row = timestamp.reshape(1, B)
    clk_row = is_click.reshape(1, B)
    ids_row = ad_ids.reshape(1, B)
    out = pl.pallas_call(
        _loss_body,
        grid=(_NB_LOSS,),
        in_specs=[
            full((B, D)),
            full((1, B)), full((1, B)), full((1, B)), full((1, B)),
            full((1, B)),
            col, col, col, col, col,
            full((256, D)), full((1, 256)),
            full((128, 256)), full((1, 128)),
            full((64, 128)), full((1, 64)),
        ],
        out_specs=full((1, 1)),
        out_shape=jax.ShapeDtypeStruct((1, 1), _F32),
        scratch_shapes=[pltpu.SMEM((2,), _F32)],
    )(ad_emb, u_row, ts_row, clk_row, ids_row, q_proba.reshape(1, B),
      u_row.reshape(B, 1), ts_row.reshape(B, 1), clk_row.reshape(B, 1),
      ids_row.reshape(B, 1), q_proba.reshape(B, 1),
      Wu1, bu1.reshape(1, -1), Wu2, bu2.reshape(1, -1),
      Wu3, bu3.reshape(1, -1))
    return out[0, 0]


def kernel(adgroup_id, cate_id, brand, user, timestamp, is_click, q_proba,
           T_adgroup, T_cate, T_brand, Wa1, ba1, Wa2, ba2, Wa3, ba3,
           Wu1, bu1, Wu2, bu2, Wu3, bu3):
    ia = adgroup_id.reshape(B).astype(jnp.int32)
    ic = cate_id.reshape(B).astype(jnp.int32)
    ib = brand.reshape(B).astype(jnp.int32)
    ec = _sc_gather_cate(ic, T_cate)
    ea, eb = _tc_gather2(ia, ib, T_adgroup.T, T_brand.T)
    ad_emb = _ad_mlp(ea, ec, eb, Wa1, ba1, Wa2, ba2, Wa3, ba3)
    return _loss(ad_emb, user.reshape(B), timestamp, is_click.astype(jnp.int32),
                 ia, q_proba, Wu1, bu1, Wu2, bu2, Wu3, bu3)


# speedup vs baseline: 2.8994x; 1.3105x over previous
import functools

import jax
import jax.numpy as jnp
from jax import lax
from jax.experimental import pallas as pl
from jax.experimental.pallas import tpu as pltpu
from jax.experimental.pallas import tpu_sc as plsc

B = 4096
D = 64
_NC = 2
_NS = 16
_NW = _NC * _NS
_BPW = B // _NW

_F32 = jnp.float32


def _dot_t(a, b):
    return lax.dot_general(a, b, (((1,), (1,)), ((), ())),
                           preferred_element_type=_F32)


def _dot(a, b):
    return lax.dot_general(a, b, (((1,), (0,)), ((), ())),
                           preferred_element_type=_F32)


def _silu(x):
    return x / (1.0 + jnp.exp(-x))


def _l2norm(x):
    n = jnp.sqrt(jnp.sum(x * x, axis=-1, keepdims=True))
    return x / jnp.maximum(n, 1e-16)


_HALF = B // 2
_CH = 512
_UNR = 8


def _sc_gather_cate(idx_c, T_c):
    mesh = plsc.ScalarSubcoreMesh(axis_name="c", num_cores=_NC)

    @functools.partial(
        pl.kernel,
        out_type=jax.ShapeDtypeStruct((B, D), _F32),
        mesh=mesh,
        scratch_types=[
            pltpu.SMEM((_CH,), jnp.int32),
            pltpu.SemaphoreType.DMA,
        ],
    )
    def gather_k(ic_h, Tc_h, oc_h, ism, sem):
        cid = lax.axis_index("c")
        base = cid * _HALF
        for ch in range(_HALF // _CH):
            off = base + ch * _CH
            pltpu.sync_copy(ic_h.at[pl.ds(off, _CH)], ism)

            def issue(j, carry, off=off):
                for k in range(_UNR):
                    pltpu.async_copy(
                        Tc_h.at[pl.ds(ism[j * _UNR + k], 1)],
                        oc_h.at[pl.ds(off + j * _UNR + k, 1)], sem)
                return carry

            lax.fori_loop(0, _CH // _UNR, issue, 0)
        pltpu.make_async_copy(Tc_h.at[pl.ds(0, _HALF)],
                              oc_h.at[pl.ds(base, _HALF)], sem).wait()

    return gather_k(idx_c, T_c)


_R_G = 256
_NB_G = B // _R_G
_LANES = 128


def _tc_gather2_body(ia_s, ib_s, ia_v, ib_v, Ta_h, Tb_h, oa, ob,
                     bufa, bufb, sa, sb):
    g = pl.program_id(0)

    def issue_block(blk, slot):
        base = blk * _R_G

        def issue(j, carry):
            for k in range(_UNR):
                r = j * _UNR + k
                ba_ = pl.multiple_of((ia_s[base + r] >> 7) * _LANES, _LANES)
                bb_ = pl.multiple_of((ib_s[base + r] >> 7) * _LANES, _LANES)
                pltpu.async_copy(Ta_h.at[:, pl.ds(ba_, _LANES)],
                                 bufa.at[slot, r], sa.at[slot])
                pltpu.async_copy(Tb_h.at[:, pl.ds(bb_, _LANES)],
                                 bufb.at[slot, r], sb.at[slot])
            return carry

        lax.fori_loop(0, _R_G // _UNR, issue, 0)

    @pl.when(g == 0)
    def _prime():
        issue_block(0, 0)

    @pl.when(g + 1 < _NB_G)
    def _next():
        issue_block(g + 1, (g + 1) % 2)

    slot = g % 2
    pltpu.make_async_copy(bufa.at[0], bufa.at[0], sa.at[slot]).wait()
    pltpu.make_async_copy(bufb.at[0], bufb.at[0], sb.at[slot]).wait()
    lane = lax.broadcasted_iota(jnp.int32, (_R_G, 1, _LANES), 2)
    ra = (ia_v[...] & (_LANES - 1)).reshape(_R_G, 1, 1)
    rb = (ib_v[...] & (_LANES - 1)).reshape(_R_G, 1, 1)
    oa[...] = jnp.sum(jnp.where(lane == ra, bufa[slot], 0.0), axis=2)
    ob[...] = jnp.sum(jnp.where(lane == rb, bufb[slot], 0.0), axis=2)


def _tc_gather2(idx_a, idx_b, T_aT, T_bT):
    row = pl.BlockSpec((_R_G, D), lambda i: (i, 0))
    col1 = pl.BlockSpec((_R_G, 1), lambda i: (i, 0))
    smem = pl.BlockSpec(memory_space=pltpu.MemorySpace.SMEM)
    return pl.pallas_call(
        _tc_gather2_body,
        grid=(_NB_G,),
        in_specs=[smem, smem, col1, col1,
                  pl.BlockSpec(memory_space=pltpu.MemorySpace.HBM),
                  pl.BlockSpec(memory_space=pltpu.MemorySpace.HBM)],
        out_specs=[row, row],
        out_shape=[jax.ShapeDtypeStruct((B, D), _F32),
                   jax.ShapeDtypeStruct((B, D), _F32)],
        scratch_shapes=[pltpu.VMEM((2, _R_G, D, _LANES), _F32),
                        pltpu.VMEM((2, _R_G, D, _LANES), _F32),
                        pltpu.SemaphoreType.DMA((2,)),
                        pltpu.SemaphoreType.DMA((2,))],
    )(idx_a, idx_b, idx_a.reshape(B, 1), idx_b.reshape(B, 1), T_aT, T_bT)


_R_MLP = 512


def _ad_mlp_body(ea, ec, eb, w1a, w1c, w1b, b1, w2, b2, w3, b3, out):
    h = _dot_t(ea[...], w1a[...]) + _dot_t(ec[...], w1c[...]) \
        + _dot_t(eb[...], w1b[...]) + b1[...]
    h = _silu(h)
    h = _silu(_dot_t(h, w2[...]) + b2[...])
    h = _dot_t(h, w3[...]) + b3[...]
    out[...] = _l2norm(_l2norm(h))


def _ad_mlp(ea, ec, eb, Wa1, ba1, Wa2, ba2, Wa3, ba3):
    full = lambda shape: pl.BlockSpec(shape, lambda i: (0, 0))
    row = pl.BlockSpec((_R_MLP, D), lambda i: (i, 0))
    return pl.pallas_call(
        _ad_mlp_body,
        grid=(B // _R_MLP,),
        in_specs=[
            row, row, row,
            full((256, D)), full((256, D)), full((256, D)), full((1, 256)),
            full((128, 256)), full((1, 128)),
            full((64, 128)), full((1, 64)),
        ],
        out_specs=row,
        out_shape=jax.ShapeDtypeStruct((B, D), _F32),
    )(ea, ec, eb, Wa1[:, 0:D], Wa1[:, D:2 * D], Wa1[:, 2 * D:3 * D],
      ba1.reshape(1, -1), Wa2, ba2.reshape(1, -1), Wa3, ba3.reshape(1, -1))


_R_LOSS = 256
_NB_LOSS = B // _R_LOSS


def _loss_body(ad_emb, u_row, ts_row, clk_row, ids_row, qp_row,
               u_col, ts_col, clk_col, ids_col, qp_col,
               w1, b1, w2, b2, w3, b3, out, acc):
    i = pl.program_id(0)

    @pl.when(i == 0)
    def _init():
        acc[0] = 0.0
        acc[1] = 0.0

    ad = ad_emb[...]
    krow = 2 * u_row[...] + clk_row[...]
    kcol = 2 * u_col[...] + 1
    maskf = ((kcol == krow) & (ts_col[...] > ts_row[...])).astype(_F32)
    msum = jnp.sum(maskf, axis=1, keepdims=True)
    hist = _dot(maskf, ad) / (msum + 1e-16)
    hist_n2 = jnp.sum(hist * hist, axis=1, keepdims=True)
    g = _silu(_dot_t(hist, w1[...]) + b1[...])
    g = _silu(_dot_t(g, w2[...]) + b2[...])
    g = _dot_t(g, w3[...]) + b3[...]
    x = _l2norm(_l2norm(g))
    user_emb = jnp.where(hist_n2 == 0.0, 0.0, x)
    u_n2 = jnp.sum(user_emb * user_emb, axis=1, keepdims=True)
    validf = ((clk_col[...] == 1) & (u_n2 != 0.0)).astype(_F32)
    logits = _dot_t(user_emb, ad) - jnp.log(qp_row[...])
    cols = lax.broadcasted_iota(jnp.int32, (_R_LOSS, B), 1)
    rows = lax.broadcasted_iota(jnp.int32, (_R_LOSS, B), 0) + i * _R_LOSS
    eye = cols == rows
    acc_hits = (ids_col[...] == ids_row[...]) != eye
    logits = jnp.where(acc_hits, -1e9, logits)
    m = jnp.max(logits, axis=1, keepdims=True)
    lse = m + jnp.log(jnp.sum(jnp.exp(logits - m), axis=1, keepdims=True))
    ad_blk = ad_emb[pl.ds(i * _R_LOSS, _R_LOSS), :]
    diag = jnp.sum(user_emb * ad_blk, axis=1, keepdims=True) \
        - jnp.log(qp_col[...])
    pos_logp = diag - lse
    acc[0] += jnp.sum(pos_logp * validf)
    acc[1] += jnp.sum(validf)

    @pl.when(i == _NB_LOSS - 1)
    def _fin():
        out[...] = jnp.broadcast_to(-acc[0] / (acc[1] + 1e-16), (1, 1))


def _loss(ad_emb, user, timestamp, is_click, ad_ids, q_proba,
          Wu1, bu1, Wu2, bu2, Wu3, bu3):
    full = lambda shape: pl.BlockSpec(shape, lambda i: (0, 0))
    col = pl.BlockSpec((_R_LOSS, 1), lambda i: (i, 0))
    u_row = user.reshape(1, B)
    ts_row = timestamp.reshape(1, B)
    clk_row = is_click.reshape(1, B)
    ids_row = ad_ids.reshape(1, B)
    out = pl.pallas_call(
        _loss_body,
        grid=(_NB_LOSS,),
        in_specs=[
            full((B, D)),
            full((1, B)), full((1, B)), full((1, B)), full((1, B)),
            full((1, B)),
            col, col, col, col, col,
            full((256, D)), full((1, 256)),
            full((128, 256)), full((1, 128)),
            full((64, 128)), full((1, 64)),
        ],
        out_specs=full((1, 1)),
        out_shape=jax.ShapeDtypeStruct((1, 1), _F32),
        scratch_shapes=[pltpu.SMEM((2,), _F32)],
    )(ad_emb, u_row, ts_row, clk_row, ids_row, q_proba.reshape(1, B),
      u_row.reshape(B, 1), ts_row.reshape(B, 1), clk_row.reshape(B, 1),
      ids_row.reshape(B, 1), q_proba.reshape(B, 1),
      Wu1, bu1.reshape(1, -1), Wu2, bu2.reshape(1, -1),
      Wu3, bu3.reshape(1, -1))
    return out[0, 0]


def kernel(adgroup_id, cate_id, brand, user, timestamp, is_click, q_proba,
           T_adgroup, T_cate, T_brand, Wa1, ba1, Wa2, ba2, Wa3, ba3,
           Wu1, bu1, Wu2, bu2, Wu3, bu3):
    ia = adgroup_id.reshape(B).astype(jnp.int32)
    ic = cate_id.reshape(B).astype(jnp.int32)
    ib = brand.reshape(B).astype(jnp.int32)
    ec = _sc_gather_cate(ic, T_cate)
    ea, eb = _tc_gather2(ia, ib, T_adgroup.T, T_brand.T)
    ad_emb = _ad_mlp(ea, ec, eb, Wa1, ba1, Wa2, ba2, Wa3, ba3)
    return _loss(ad_emb, user.reshape(B), timestamp, is_click.astype(jnp.int32),
                 ia, q_proba, Wu1, bu1, Wu2, bu2, Wu3, bu3)


# speedup vs baseline: 2.9841x; 1.0292x over previous
import functools

import jax
import jax.numpy as jnp
from jax import lax
from jax.experimental import pallas as pl
from jax.experimental.pallas import tpu as pltpu
from jax.experimental.pallas import tpu_sc as plsc

B = 4096
D = 64
_NC = 2
_NS = 16
_NW = _NC * _NS
_BPW = B // _NW

_F32 = jnp.float32


def _dot_t(a, b):
    return lax.dot_general(a, b, (((1,), (1,)), ((), ())),
                           preferred_element_type=_F32)


def _dot(a, b):
    return lax.dot_general(a, b, (((1,), (0,)), ((), ())),
                           preferred_element_type=_F32)


def _silu(x):
    return x / (1.0 + jnp.exp(-x))


def _l2norm(x):
    n = jnp.sqrt(jnp.sum(x * x, axis=-1, keepdims=True))
    return x / jnp.maximum(n, 1e-16)


_HALF = B // 2
_CH = 512
_UNR = 8


def _sc_gather_cate(idx_c, T_c):
    mesh = plsc.ScalarSubcoreMesh(axis_name="c", num_cores=_NC)

    @functools.partial(
        pl.kernel,
        out_type=jax.ShapeDtypeStruct((B, D), _F32),
        mesh=mesh,
        scratch_types=[
            pltpu.SMEM((_CH,), jnp.int32),
            pltpu.SemaphoreType.DMA,
        ],
    )
    def gather_k(ic_h, Tc_h, oc_h, ism, sem):
        cid = lax.axis_index("c")
        base = cid * _HALF
        for ch in range(_HALF // _CH):
            off = base + ch * _CH
            pltpu.sync_copy(ic_h.at[pl.ds(off, _CH)], ism)

            def issue(j, carry, off=off):
                for k in range(_UNR):
                    pltpu.async_copy(
                        Tc_h.at[pl.ds(ism[j * _UNR + k], 1)],
                        oc_h.at[pl.ds(off + j * _UNR + k, 1)], sem)
                return carry

            lax.fori_loop(0, _CH // _UNR, issue, 0)
        pltpu.make_async_copy(Tc_h.at[pl.ds(0, _HALF)],
                              oc_h.at[pl.ds(base, _HALF)], sem).wait()

    return gather_k(idx_c, T_c)


_R_G = 256
_NB_G = B // _R_G
_LANES = 128


def _tc_gather2_body(ia_s, ib_s, iv2, Ta_h, Tb_h, oa, ob,
                     bufa, bufb, sa, sb):
    g = pl.program_id(0)

    def issue_block(blk, slot):
        base = blk * _R_G

        def issue(j, carry):
            for k in range(_UNR):
                r = j * _UNR + k
                ba_ = pl.multiple_of((ia_s[base + r] >> 7) * _LANES, _LANES)
                bb_ = pl.multiple_of((ib_s[base + r] >> 7) * _LANES, _LANES)
                pltpu.async_copy(Ta_h.at[:, pl.ds(ba_, _LANES)],
                                 bufa.at[slot, r], sa.at[slot])
                pltpu.async_copy(Tb_h.at[:, pl.ds(bb_, _LANES)],
                                 bufb.at[slot, r], sb.at[slot])
            return carry

        lax.fori_loop(0, _R_G // _UNR, issue, 0)

    @pl.when(g == 0)
    def _prime():
        issue_block(0, 0)

    @pl.when(g + 1 < _NB_G)
    def _next():
        issue_block(g + 1, (g + 1) % 2)

    slot = g % 2
    pltpu.make_async_copy(bufa.at[0], bufa.at[0], sa.at[slot]).wait()
    pltpu.make_async_copy(bufb.at[0], bufb.at[0], sb.at[slot]).wait()
    lane = lax.broadcasted_iota(jnp.int32, (_R_G, 1, _LANES), 2)
    ra = (iv2[:, 0:1] & (_LANES - 1)).reshape(_R_G, 1, 1)
    rb = (iv2[:, 1:2] & (_LANES - 1)).reshape(_R_G, 1, 1)
    oa[...] = jnp.sum(jnp.where(lane == ra, bufa[slot], 0.0), axis=2)
    ob[...] = jnp.sum(jnp.where(lane == rb, bufb[slot], 0.0), axis=2)


def _tc_gather2(idx_a, idx_b, T_aT, T_bT):
    row = pl.BlockSpec((_R_G, D), lambda i: (i, 0))
    col2 = pl.BlockSpec((_R_G, 2), lambda i: (i, 0))
    smem = pl.BlockSpec(memory_space=pltpu.MemorySpace.SMEM)
    return pl.pallas_call(
        _tc_gather2_body,
        grid=(_NB_G,),
        in_specs=[smem, smem, col2,
                  pl.BlockSpec(memory_space=pltpu.MemorySpace.HBM),
                  pl.BlockSpec(memory_space=pltpu.MemorySpace.HBM)],
        out_specs=[row, row],
        out_shape=[jax.ShapeDtypeStruct((B, D), _F32),
                   jax.ShapeDtypeStruct((B, D), _F32)],
        scratch_shapes=[pltpu.VMEM((2, _R_G, D, _LANES), _F32),
                        pltpu.VMEM((2, _R_G, D, _LANES), _F32),
                        pltpu.SemaphoreType.DMA((2,)),
                        pltpu.SemaphoreType.DMA((2,))],
    )(idx_a, idx_b, jnp.stack([idx_a, idx_b], axis=1), T_aT, T_bT)


_R_MLP = 512


def _ad_mlp_body(ea, ec, eb, w1a, w1c, w1b, b1, w2, b2, w3, b3, out):
    h = _dot_t(ea[...], w1a[...]) + _dot_t(ec[...], w1c[...]) \
        + _dot_t(eb[...], w1b[...]) + b1[...]
    h = _silu(h)
    h = _silu(_dot_t(h, w2[...]) + b2[...])
    h = _dot_t(h, w3[...]) + b3[...]
    out[...] = _l2norm(_l2norm(h))


def _ad_mlp(ea, ec, eb, Wa1, ba1, Wa2, ba2, Wa3, ba3):
    full = lambda shape: pl.BlockSpec(shape, lambda i: (0, 0))
    row = pl.BlockSpec((_R_MLP, D), lambda i: (i, 0))
    return pl.pallas_call(
        _ad_mlp_body,
        grid=(B // _R_MLP,),
        in_specs=[
            row, row, row,
            full((256, D)), full((256, D)), full((256, D)), full((1, 256)),
            full((128, 256)), full((1, 128)),
            full((64, 128)), full((1, 64)),
        ],
        out_specs=row,
        out_shape=jax.ShapeDtypeStruct((B, D), _F32),
    )(ea, ec, eb, Wa1[:, 0:D], Wa1[:, D:2 * D], Wa1[:, 2 * D:3 * D],
      ba1.reshape(1, -1), Wa2, ba2.reshape(1, -1), Wa3, ba3.reshape(1, -1))


_R_LOSS = 256
_NB_LOSS = B // _R_LOSS


def _loss_body(ad_emb, u_row, ts_row, clk_row, ids_row, qp_row, cols5,
               w1, b1, w2, b2, w3, b3, out, acc):
    i = pl.program_id(0)

    @pl.when(i == 0)
    def _init():
        acc[0] = 0.0
        acc[1] = 0.0

    ad = ad_emb[...]
    u_col = cols5[:, 0:1]
    clk_col = cols5[:, 1:2]
    ids_col = cols5[:, 2:3]
    ts_col = lax.bitcast_convert_type(cols5[:, 3:4], _F32)
    qp_col = lax.bitcast_convert_type(cols5[:, 4:5], _F32)
    krow = 2 * u_row[...] + clk_row[...]
    kcol = 2 * u_col + 1
    maskf = ((kcol == krow) & (ts_col > ts_row[...])).astype(_F32)
    msum = jnp.sum(maskf, axis=1, keepdims=True)
    hist = _dot(maskf, ad) / (msum + 1e-16)
    hist_n2 = jnp.sum(hist * hist, axis=1, keepdims=True)
    g = _silu(_dot_t(hist, w1[...]) + b1[...])
    g = _silu(_dot_t(g, w2[...]) + b2[...])
    g = _dot_t(g, w3[...]) + b3[...]
    x = _l2norm(_l2norm(g))
    user_emb = jnp.where(hist_n2 == 0.0, 0.0, x)
    u_n2 = jnp.sum(user_emb * user_emb, axis=1, keepdims=True)
    validf = ((clk_col == 1) & (u_n2 != 0.0)).astype(_F32)
    logits = _dot_t(user_emb, ad) - jnp.log(qp_row[...])
    cols = lax.broadcasted_iota(jnp.int32, (_R_LOSS, B), 1)
    rows = lax.broadcasted_iota(jnp.int32, (_R_LOSS, B), 0) + i * _R_LOSS
    eye = cols == rows
    acc_hits = (ids_col == ids_row[...]) != eye
    logits = jnp.where(acc_hits, -1e9, logits)
    m = jnp.max(logits, axis=1, keepdims=True)
    lse = m + jnp.log(jnp.sum(jnp.exp(logits - m), axis=1, keepdims=True))
    ad_blk = ad_emb[pl.ds(i * _R_LOSS, _R_LOSS), :]
    diag = jnp.sum(user_emb * ad_blk, axis=1, keepdims=True) \
        - jnp.log(qp_col)
    pos_logp = diag - lse
    acc[0] += jnp.sum(pos_logp * validf)
    acc[1] += jnp.sum(validf)

    @pl.when(i == _NB_LOSS - 1)
    def _fin():
        out[...] = jnp.broadcast_to(-acc[0] / (acc[1] + 1e-16), (1, 1))


def _loss(ad_emb, user, timestamp, is_click, ad_ids, q_proba,
          Wu1, bu1, Wu2, bu2, Wu3, bu3):
    full = lambda shape: pl.BlockSpec(shape, lambda i: (0, 0))
    col5 = pl.BlockSpec((_R_LOSS, 5), lambda i: (i, 0))
    u_row = user.reshape(1, B)
    ts_row = timestamp.reshape(1, B)
    clk_row = is_click.reshape(1, B)
    ids_row = ad_ids.reshape(1, B)
    out = pl.pallas_call(
        _loss_body,
        grid=(_NB_LOSS,),
        in_specs=[
            full((B, D)),
            full((1, B)), full((1, B)), full((1, B)), full((1, B)),
            full((1, B)),
            col5,
            full((256, D)), full((1, 256)),
            full((128, 256)), full((1, 128)),
            full((64, 128)), full((1, 64)),
        ],
        out_specs=full((1, 1)),
        out_shape=jax.ShapeDtypeStruct((1, 1), _F32),
        scratch_shapes=[pltpu.SMEM((2,), _F32)],
    )(ad_emb, u_row, ts_row, clk_row, ids_row, q_proba.reshape(1, B),
      jnp.stack([user, is_click,
                 ad_ids, lax.bitcast_convert_type(timestamp, jnp.int32),
                 lax.bitcast_convert_type(q_proba.reshape(B), jnp.int32)],
                axis=1),
      Wu1, bu1.reshape(1, -1), Wu2, bu2.reshape(1, -1),
      Wu3, bu3.reshape(1, -1))
    return out[0, 0]


def kernel(adgroup_id, cate_id, brand, user, timestamp, is_click, q_proba,
           T_adgroup, T_cate, T_brand, Wa1, ba1, Wa2, ba2, Wa3, ba3,
           Wu1, bu1, Wu2, bu2, Wu3, bu3):
    ia = adgroup_id.reshape(B).astype(jnp.int32)
    ic = cate_id.reshape(B).astype(jnp.int32)
    ib = brand.reshape(B).astype(jnp.int32)
    ec = _sc_gather_cate(ic, T_cate)
    ea, eb = _tc_gather2(ia, ib, T_adgroup.T, T_brand.T)
    ad_emb = _ad_mlp(ea, ec, eb, Wa1, ba1, Wa2, ba2, Wa3, ba3)
    return _loss(ad_emb, user.reshape(B), timestamp, is_click.astype(jnp.int32),
                 ia, q_proba, Wu1, bu1, Wu2, bu2, Wu3, bu3)


# speedup vs baseline: 3.0442x; 1.0201x over previous
import functools

import jax
import jax.numpy as jnp
from jax import lax
from jax.experimental import pallas as pl
from jax.experimental.pallas import tpu as pltpu
from jax.experimental.pallas import tpu_sc as plsc

B = 4096
D = 64
_NC = 2
_NS = 16
_NW = _NC * _NS
_BPW = B // _NW

_F32 = jnp.float32


def _dot_t(a, b):
    return lax.dot_general(a, b, (((1,), (1,)), ((), ())),
                           preferred_element_type=_F32)


def _dot(a, b):
    return lax.dot_general(a, b, (((1,), (0,)), ((), ())),
                           preferred_element_type=_F32)


def _silu(x):
    return x / (1.0 + jnp.exp(-x))


def _l2norm(x):
    n = jnp.sqrt(jnp.sum(x * x, axis=-1, keepdims=True))
    return x / jnp.maximum(n, 1e-16)


_HALF = B // 2
_CH = 512
_UNR = 8


def _sc_gather_cate(idx_c, T_c):
    mesh = plsc.ScalarSubcoreMesh(axis_name="c", num_cores=_NC)

    @functools.partial(
        pl.kernel,
        out_type=jax.ShapeDtypeStruct((B, D), _F32),
        mesh=mesh,
        scratch_types=[
            pltpu.SMEM((_CH,), jnp.int32),
            pltpu.SemaphoreType.DMA,
        ],
    )
    def gather_k(ic_h, Tc_h, oc_h, ism, sem):
        cid = lax.axis_index("c")
        base = cid * _HALF
        for ch in range(_HALF // _CH):
            off = base + ch * _CH
            pltpu.sync_copy(ic_h.at[pl.ds(off, _CH)], ism)

            def issue(j, carry, off=off):
                for k in range(_UNR):
                    pltpu.async_copy(
                        Tc_h.at[pl.ds(ism[j * _UNR + k], 1)],
                        oc_h.at[pl.ds(off + j * _UNR + k, 1)], sem)
                return carry

            lax.fori_loop(0, _CH // _UNR, issue, 0)
        pltpu.make_async_copy(Tc_h.at[pl.ds(0, _HALF)],
                              oc_h.at[pl.ds(base, _HALF)], sem).wait()

    return gather_k(idx_c, T_c)


_R_G = 256
_NB_G = B // _R_G
_LANES = 128


def _tc_gather2_body(ia_s, ib_s, iv2, Ta_h, Tb_h, oa, ob,
                     bufa, bufb, sa, sb):
    g = pl.program_id(0)

    def issue_block(blk, slot):
        base = blk * _R_G

        def issue(j, carry):
            for k in range(_UNR):
                r = j * _UNR + k
                ba_ = pl.multiple_of((ia_s[base + r] >> 7) * _LANES, _LANES)
                bb_ = pl.multiple_of((ib_s[base + r] >> 7) * _LANES, _LANES)
                pltpu.async_copy(Ta_h.at[:, pl.ds(ba_, _LANES)],
                                 bufa.at[slot, r], sa.at[slot])
                pltpu.async_copy(Tb_h.at[:, pl.ds(bb_, _LANES)],
                                 bufb.at[slot, r], sb.at[slot])
            return carry

        lax.fori_loop(0, _R_G // _UNR, issue, 0)

    @pl.when(g == 0)
    def _prime():
        issue_block(0, 0)
        issue_block(1, 1)

    @pl.when(g + 2 < _NB_G)
    def _next():
        issue_block(g + 2, (g + 2) % 3)

    slot = g % 3
    pltpu.make_async_copy(bufa.at[0], bufa.at[0], sa.at[slot]).wait()
    pltpu.make_async_copy(bufb.at[0], bufb.at[0], sb.at[slot]).wait()
    lane = lax.broadcasted_iota(jnp.int32, (_R_G, 1, _LANES), 2)
    ra = (iv2[:, 0:1] & (_LANES - 1)).reshape(_R_G, 1, 1)
    rb = (iv2[:, 1:2] & (_LANES - 1)).reshape(_R_G, 1, 1)
    oa[...] = jnp.sum(jnp.where(lane == ra, bufa[slot], 0.0), axis=2)
    ob[...] = jnp.sum(jnp.where(lane == rb, bufb[slot], 0.0), axis=2)


def _tc_gather2(idx_a, idx_b, T_aT, T_bT):
    row = pl.BlockSpec((_R_G, D), lambda i: (i, 0))
    col2 = pl.BlockSpec((_R_G, 2), lambda i: (i, 0))
    smem = pl.BlockSpec(memory_space=pltpu.MemorySpace.SMEM)
    return pl.pallas_call(
        _tc_gather2_body,
        grid=(_NB_G,),
        in_specs=[smem, smem, col2,
                  pl.BlockSpec(memory_space=pltpu.MemorySpace.HBM),
                  pl.BlockSpec(memory_space=pltpu.MemorySpace.HBM)],
        out_specs=[row, row],
        out_shape=[jax.ShapeDtypeStruct((B, D), _F32),
                   jax.ShapeDtypeStruct((B, D), _F32)],
        scratch_shapes=[pltpu.VMEM((3, _R_G, D, _LANES), _F32),
                        pltpu.VMEM((3, _R_G, D, _LANES), _F32),
                        pltpu.SemaphoreType.DMA((3,)),
                        pltpu.SemaphoreType.DMA((3,))],
    )(idx_a, idx_b, jnp.stack([idx_a, idx_b], axis=1), T_aT, T_bT)


_R_MLP = 512


def _ad_mlp_body(ea, ec, eb, w1a, w1c, w1b, b1, w2, b2, w3, b3, out):
    h = _dot_t(ea[...], w1a[...]) + _dot_t(ec[...], w1c[...]) \
        + _dot_t(eb[...], w1b[...]) + b1[...]
    h = _silu(h)
    h = _silu(_dot_t(h, w2[...]) + b2[...])
    h = _dot_t(h, w3[...]) + b3[...]
    out[...] = _l2norm(_l2norm(h))


def _ad_mlp(ea, ec, eb, Wa1, ba1, Wa2, ba2, Wa3, ba3):
    full = lambda shape: pl.BlockSpec(shape, lambda i: (0, 0))
    row = pl.BlockSpec((_R_MLP, D), lambda i: (i, 0))
    return pl.pallas_call(
        _ad_mlp_body,
        grid=(B // _R_MLP,),
        in_specs=[
            row, row, row,
            full((256, D)), full((256, D)), full((256, D)), full((1, 256)),
            full((128, 256)), full((1, 128)),
            full((64, 128)), full((1, 64)),
        ],
        out_specs=row,
        out_shape=jax.ShapeDtypeStruct((B, D), _F32),
    )(ea, ec, eb, Wa1[:, 0:D], Wa1[:, D:2 * D], Wa1[:, 2 * D:3 * D],
      ba1.reshape(1, -1), Wa2, ba2.reshape(1, -1), Wa3, ba3.reshape(1, -1))


_R_LOSS = 512
_NB_LOSS = B // _R_LOSS


def _loss_body(ad_emb, u_row, ts_row, clk_row, ids_row, qp_row, cols5,
               w1, b1, w2, b2, w3, b3, out, acc):
    i = pl.program_id(0)

    @pl.when(i == 0)
    def _init():
        acc[0] = 0.0
        acc[1] = 0.0

    ad = ad_emb[...]
    u_col = cols5[:, 0:1]
    clk_col = cols5[:, 1:2]
    ids_col = cols5[:, 2:3]
    ts_col = lax.bitcast_convert_type(cols5[:, 3:4], _F32)
    qp_col = lax.bitcast_convert_type(cols5[:, 4:5], _F32)
    krow = 2 * u_row[...] + clk_row[...]
    kcol = 2 * u_col + 1
    maskf = ((kcol == krow) & (ts_col > ts_row[...])).astype(_F32)
    msum = jnp.sum(maskf, axis=1, keepdims=True)
    hist = _dot(maskf, ad) / (msum + 1e-16)
    hist_n2 = jnp.sum(hist * hist, axis=1, keepdims=True)
    g = _silu(_dot_t(hist, w1[...]) + b1[...])
    g = _silu(_dot_t(g, w2[...]) + b2[...])
    g = _dot_t(g, w3[...]) + b3[...]
    x = _l2norm(_l2norm(g))
    user_emb = jnp.where(hist_n2 == 0.0, 0.0, x)
    u_n2 = jnp.sum(user_emb * user_emb, axis=1, keepdims=True)
    validf = ((clk_col == 1) & (u_n2 != 0.0)).astype(_F32)
    logits = _dot_t(user_emb, ad) - jnp.log(qp_row[...])
    cols = lax.broadcasted_iota(jnp.int32, (_R_LOSS, B), 1)
    rows = lax.broadcasted_iota(jnp.int32, (_R_LOSS, B), 0) + i * _R_LOSS
    eye = cols == rows
    acc_hits = (ids_col == ids_row[...]) != eye
    logits = jnp.where(acc_hits, -1e9, logits)
    m = jnp.max(logits, axis=1, keepdims=True)
    lse = m + jnp.log(jnp.sum(jnp.exp(logits - m), axis=1, keepdims=True))
    ad_blk = ad_emb[pl.ds(i * _R_LOSS, _R_LOSS), :]
    diag = jnp.sum(user_emb * ad_blk, axis=1, keepdims=True) \
        - jnp.log(qp_col)
    pos_logp = diag - lse
    acc[0] += jnp.sum(pos_logp * validf)
    acc[1] += jnp.sum(validf)

    @pl.when(i == _NB_LOSS - 1)
    def _fin():
        out[...] = jnp.broadcast_to(-acc[0] / (acc[1] + 1e-16), (1, 1))


def _loss(ad_emb, user, timestamp, is_click, ad_ids, q_proba,
          Wu1, bu1, Wu2, bu2, Wu3, bu3):
    full = lambda shape: pl.BlockSpec(shape, lambda i: (0, 0))
    col5 = pl.BlockSpec((_R_LOSS, 5), lambda i: (i, 0))
    u_row = user.reshape(1, B)
    ts_row = timestamp.reshape(1, B)
    clk_row = is_click.reshape(1, B)
    ids_row = ad_ids.reshape(1, B)
    out = pl.pallas_call(
        _loss_body,
        grid=(_NB_LOSS,),
        in_specs=[
            full((B, D)),
            full((1, B)), full((1, B)), full((1, B)), full((1, B)),
            full((1, B)),
            col5,
            full((256, D)), full((1, 256)),
            full((128, 256)), full((1, 128)),
            full((64, 128)), full((1, 64)),
        ],
        out_specs=full((1, 1)),
        out_shape=jax.ShapeDtypeStruct((1, 1), _F32),
        scratch_shapes=[pltpu.SMEM((2,), _F32)],
    )(ad_emb, u_row, ts_row, clk_row, ids_row, q_proba.reshape(1, B),
      jnp.stack([user, is_click,
                 ad_ids, lax.bitcast_convert_type(timestamp, jnp.int32),
                 lax.bitcast_convert_type(q_proba.reshape(B), jnp.int32)],
                axis=1),
      Wu1, bu1.reshape(1, -1), Wu2, bu2.reshape(1, -1),
      Wu3, bu3.reshape(1, -1))
    return out[0, 0]


def kernel(adgroup_id, cate_id, brand, user, timestamp, is_click, q_proba,
           T_adgroup, T_cate, T_brand, Wa1, ba1, Wa2, ba2, Wa3, ba3,
           Wu1, bu1, Wu2, bu2, Wu3, bu3):
    ia = adgroup_id.reshape(B).astype(jnp.int32)
    ic = cate_id.reshape(B).astype(jnp.int32)
    ib = brand.reshape(B).astype(jnp.int32)
    ec = _sc_gather_cate(ic, T_cate)
    ea, eb = _tc_gather2(ia, ib, T_adgroup.T, T_brand.T)
    ad_emb = _ad_mlp(ea, ec, eb, Wa1, ba1, Wa2, ba2, Wa3, ba3)
    return _loss(ad_emb, user.reshape(B), timestamp, is_click.astype(jnp.int32),
                 ia, q_proba, Wu1, bu1, Wu2, bu2, Wu3, bu3)


# speedup vs baseline: 3.0583x; 1.0046x over previous
import functools

import jax
import jax.numpy as jnp
from jax import lax
from jax.experimental import pallas as pl
from jax.experimental.pallas import tpu as pltpu
from jax.experimental.pallas import tpu_sc as plsc

B = 4096
D = 64
_NC = 2

_F32 = jnp.float32


def _dot_t(a, b):
    return lax.dot_general(a, b, (((1,), (1,)), ((), ())),
                           preferred_element_type=_F32)


def _dot(a, b):
    return lax.dot_general(a, b, (((1,), (0,)), ((), ())),
                           preferred_element_type=_F32)


def _silu(x):
    return x / (1.0 + jnp.exp(-x))


def _l2norm(x):
    n = jnp.sqrt(jnp.sum(x * x, axis=-1, keepdims=True))
    return x / jnp.maximum(n, 1e-16)


_HALF = B // 2
_CH = 512
_UNR = 8


def _sc_gather_cate(idx_c, T_c):
    mesh = plsc.ScalarSubcoreMesh(axis_name="c", num_cores=_NC)

    @functools.partial(
        pl.kernel,
        out_type=jax.ShapeDtypeStruct((B, D), _F32),
        mesh=mesh,
        scratch_types=[
            pltpu.SMEM((_CH,), jnp.int32),
            pltpu.SemaphoreType.DMA,
        ],
    )
    def gather_k(ic_h, Tc_h, oc_h, ism, sem):
        cid = lax.axis_index("c")
        base = cid * _HALF
        for ch in range(_HALF // _CH):
            off = base + ch * _CH
            pltpu.sync_copy(ic_h.at[pl.ds(off, _CH)], ism)

            def issue(j, carry, off=off):
                for k in range(_UNR):
                    pltpu.async_copy(
                        Tc_h.at[pl.ds(ism[j * _UNR + k], 1)],
                        oc_h.at[pl.ds(off + j * _UNR + k, 1)], sem)
                return carry

            lax.fori_loop(0, _CH // _UNR, issue, 0)
        pltpu.make_async_copy(Tc_h.at[pl.ds(0, _HALF)],
                              oc_h.at[pl.ds(base, _HALF)], sem).wait()

    return gather_k(idx_c, T_c)


_R_G = 256
_NB_G = B // _R_G
_LANES = 128


def _tc_gather2_body(ia_s, ib_s, iv2, Ta_h, Tb_h, oa, ob,
                     bufa, bufb, sa, sb):
    g = pl.program_id(0)

    def issue_block(blk, slot):
        base = blk * _R_G

        def issue(j, carry):
            for k in range(_UNR):
                r = j * _UNR + k
                ba_ = pl.multiple_of((ia_s[base + r] >> 7) * _LANES, _LANES)
                bb_ = pl.multiple_of((ib_s[base + r] >> 7) * _LANES, _LANES)
                pltpu.async_copy(Ta_h.at[:, pl.ds(ba_, _LANES)],
                                 bufa.at[slot, r], sa.at[slot])
                pltpu.async_copy(Tb_h.at[:, pl.ds(bb_, _LANES)],
                                 bufb.at[slot, r], sb.at[slot])
            return carry

        lax.fori_loop(0, _R_G // _UNR, issue, 0)

    @pl.when(g == 0)
    def _prime():
        issue_block(0, 0)
        issue_block(1, 1)

    @pl.when(g + 2 < _NB_G)
    def _next():
        issue_block(g + 2, (g + 2) % 3)

    slot = g % 3
    pltpu.make_async_copy(bufa.at[0], bufa.at[0], sa.at[slot]).wait()
    pltpu.make_async_copy(bufb.at[0], bufb.at[0], sb.at[slot]).wait()
    lane = lax.broadcasted_iota(jnp.int32, (_R_G, 1, _LANES), 2)
    ra = (iv2[:, 0:1] & (_LANES - 1)).reshape(_R_G, 1, 1)
    rb = (iv2[:, 1:2] & (_LANES - 1)).reshape(_R_G, 1, 1)
    oa[...] = jnp.sum(jnp.where(lane == ra, bufa[slot], 0.0), axis=2)
    ob[...] = jnp.sum(jnp.where(lane == rb, bufb[slot], 0.0), axis=2)


def _tc_gather2(idx_a, idx_b, T_aT, T_bT):
    row = pl.BlockSpec((_R_G, D), lambda i: (i, 0))
    col2 = pl.BlockSpec((_R_G, 2), lambda i: (i, 0))
    smem = pl.BlockSpec(memory_space=pltpu.MemorySpace.SMEM)
    return pl.pallas_call(
        _tc_gather2_body,
        grid=(_NB_G,),
        in_specs=[smem, smem, col2,
                  pl.BlockSpec(memory_space=pltpu.MemorySpace.HBM),
                  pl.BlockSpec(memory_space=pltpu.MemorySpace.HBM)],
        out_specs=[row, row],
        out_shape=[jax.ShapeDtypeStruct((B, D), _F32),
                   jax.ShapeDtypeStruct((B, D), _F32)],
        scratch_shapes=[pltpu.VMEM((3, _R_G, D, _LANES), _F32),
                        pltpu.VMEM((3, _R_G, D, _LANES), _F32),
                        pltpu.SemaphoreType.DMA((3,)),
                        pltpu.SemaphoreType.DMA((3,))],
    )(idx_a, idx_b, jnp.stack([idx_a, idx_b], axis=1), T_aT, T_bT)


_R_MLP = 512


def _ad_mlp_body(ea, ec, eb, w1a, w1c, w1b, b1, w2, b2, w3, b3, out):
    h = _dot_t(ea[...], w1a[...]) + _dot_t(ec[...], w1c[...]) \
        + _dot_t(eb[...], w1b[...]) + b1[...]
    h = _silu(h)
    h = _silu(_dot_t(h, w2[...]) + b2[...])
    h = _dot_t(h, w3[...]) + b3[...]
    out[...] = _l2norm(_l2norm(h))


def _ad_mlp(ea, ec, eb, Wa1, ba1, Wa2, ba2, Wa3, ba3):
    full = lambda shape: pl.BlockSpec(shape, lambda i: (0, 0))
    row = pl.BlockSpec((_R_MLP, D), lambda i: (i, 0))
    return pl.pallas_call(
        _ad_mlp_body,
        grid=(B // _R_MLP,),
        in_specs=[
            row, row, row,
            full((256, D)), full((256, D)), full((256, D)), full((1, 256)),
            full((128, 256)), full((1, 128)),
            full((64, 128)), full((1, 64)),
        ],
        out_specs=row,
        out_shape=jax.ShapeDtypeStruct((B, D), _F32),
    )(ea, ec, eb, Wa1[:, 0:D], Wa1[:, D:2 * D], Wa1[:, 2 * D:3 * D],
      ba1.reshape(1, -1), Wa2, ba2.reshape(1, -1), Wa3, ba3.reshape(1, -1))


_R_LOSS = 512
_NB_LOSS = B // _R_LOSS


def _loss_body(ad_emb, u_row, ts_row, clk_row, ids_row, qp_row, cols5,
               w1, b1, w2, b2, w3, b3, out, acc):
    i = pl.program_id(0)

    @pl.when(i == 0)
    def _init():
        acc[0] = 0.0
        acc[1] = 0.0

    ad = ad_emb[...]
    u_col = cols5[:, 0:1]
    clk_col = cols5[:, 1:2]
    ids_col = cols5[:, 2:3]
    ts_col = lax.bitcast_convert_type(cols5[:, 3:4], _F32)
    qp_col = lax.bitcast_convert_type(cols5[:, 4:5], _F32)
    krow = 2 * u_row[...] + clk_row[...]
    kcol = 2 * u_col + 1
    maskf = ((kcol == krow) & (ts_col > ts_row[...])).astype(_F32)
    msum = jnp.sum(maskf, axis=1, keepdims=True)
    hist = _dot(maskf, ad) / (msum + 1e-16)
    hist_n2 = jnp.sum(hist * hist, axis=1, keepdims=True)
    g = _silu(_dot_t(hist, w1[...]) + b1[...])
    g = _silu(_dot_t(g, w2[...]) + b2[...])
    g = _dot_t(g, w3[...]) + b3[...]
    x = _l2norm(_l2norm(g))
    user_emb = jnp.where(hist_n2 == 0.0, 0.0, x)
    u_n2 = jnp.sum(user_emb * user_emb, axis=1, keepdims=True)
    validf = ((clk_col == 1) & (u_n2 != 0.0)).astype(_F32)
    logits = _dot_t(user_emb, ad) - jnp.log(qp_row[...])
    cols = lax.broadcasted_iota(jnp.int32, (_R_LOSS, B), 1)
    rows = lax.broadcasted_iota(jnp.int32, (_R_LOSS, B), 0) + i * _R_LOSS
    eye = cols == rows
    acc_hits = (ids_col == ids_row[...]) != eye
    logits = jnp.where(acc_hits, -1e9, logits)
    m = jnp.max(logits, axis=1, keepdims=True)
    lse = m + jnp.log(jnp.sum(jnp.exp(logits - m), axis=1, keepdims=True))
    ad_blk = ad_emb[pl.ds(i * _R_LOSS, _R_LOSS), :]
    diag = jnp.sum(user_emb * ad_blk, axis=1, keepdims=True) \
        - jnp.log(qp_col)
    pos_logp = diag - lse
    acc[0] += jnp.sum(pos_logp * validf)
    acc[1] += jnp.sum(validf)

    @pl.when(i == _NB_LOSS - 1)
    def _fin():
        out[...] = jnp.broadcast_to(-acc[0] / (acc[1] + 1e-16), (1, 1))


def _loss(ad_emb, user, timestamp, is_click, ad_ids, q_proba,
          Wu1, bu1, Wu2, bu2, Wu3, bu3):
    full = lambda shape: pl.BlockSpec(shape, lambda i: (0, 0))
    col5 = pl.BlockSpec((_R_LOSS, 5), lambda i: (i, 0))
    u_row = user.reshape(1, B)
    ts_row = timestamp.reshape(1, B)
    clk_row = is_click.reshape(1, B)
    ids_row = ad_ids.reshape(1, B)
    out = pl.pallas_call(
        _loss_body,
        grid=(_NB_LOSS,),
        in_specs=[
            full((B, D)),
            full((1, B)), full((1, B)), full((1, B)), full((1, B)),
            full((1, B)),
            col5,
            full((256, D)), full((1, 256)),
            full((128, 256)), full((1, 128)),
            full((64, 128)), full((1, 64)),
        ],
        out_specs=full((1, 1)),
        out_shape=jax.ShapeDtypeStruct((1, 1), _F32),
        scratch_shapes=[pltpu.SMEM((2,), _F32)],
    )(ad_emb, u_row, ts_row, clk_row, ids_row, q_proba.reshape(1, B),
      jnp.stack([user, is_click,
                 ad_ids, lax.bitcast_convert_type(timestamp, jnp.int32),
                 lax.bitcast_convert_type(q_proba.reshape(B), jnp.int32)],
                axis=1),
      Wu1, bu1.reshape(1, -1), Wu2, bu2.reshape(1, -1),
      Wu3, bu3.reshape(1, -1))
    return out[0, 0]


def kernel(adgroup_id, cate_id, brand, user, timestamp, is_click, q_proba,
           T_adgroup, T_cate, T_brand, Wa1, ba1, Wa2, ba2, Wa3, ba3,
           Wu1, bu1, Wu2, bu2, Wu3, bu3):
    ia = adgroup_id.reshape(B).astype(jnp.int32)
    ic = cate_id.reshape(B).astype(jnp.int32)
    ib = brand.reshape(B).astype(jnp.int32)
    ec = _sc_gather_cate(ic, T_cate)
    ea, eb = _tc_gather2(ia, ib, T_adgroup.T, T_brand.T)
    ad_emb = _ad_mlp(ea, ec, eb, Wa1, ba1, Wa2, ba2, Wa3, ba3)
    return _loss(ad_emb, user.reshape(B), timestamp, is_click.astype(jnp.int32),
                 ia, q_proba, Wu1, bu1, Wu2, bu2, Wu3, bu3)
